# Initial kernel scaffold; baseline (speedup 1.0000x reference)
import jax
import jax.numpy as jnp
from jax.experimental import pallas as pl


def kernel(point_cloud, params):
    raise NotImplementedError("write your pallas kernel here")


# speedup vs baseline: 20.4592x; 20.4592x over previous
import functools

import jax
import jax.numpy as jnp
import numpy as np
from jax import lax
from jax.experimental import pallas as pl
from jax.experimental.pallas import tpu as pltpu
from jax.experimental.pallas import tpu_sc as plsc

_SA_CFGS = [
    (1024, [0.05, 0.1], [16, 32]),
    (256, [0.1, 0.2], [16, 32]),
    (64, [0.2, 0.4], [16, 32]),
    (16, [0.4, 0.8], [16, 32]),
]
_B = 8
_N0 = 4096


def _fold(p):
    s = p["gamma"] / np.sqrt(1.0 + 1e-5)
    return p["W"] * s[:, None], p["b"] * s + p["beta"]


def _fps_body(x_ref, y_ref, z_ref, qx_ref, qy_ref, qz_ref, dist_ref):
    B, N = x_ref.shape
    S = qx_ref.shape[0]
    X = x_ref[...]
    Y = y_ref[...]
    Z = z_ref[...]
    iota = lax.broadcasted_iota(jnp.int32, (B, N), 1)
    eye = lax.broadcasted_iota(jnp.int32, (B, B), 0) == lax.broadcasted_iota(
        jnp.int32, (B, B), 1
    )
    dist_ref[...] = jnp.full((B, N), 1e10, jnp.float32)

    def row(v):
        return jnp.sum(jnp.where(eye, v, 0.0), axis=0, keepdims=True)

    def body(i, far):
        onehot = iota == far
        cx = jnp.sum(jnp.where(onehot, X, 0.0), axis=1, keepdims=True)
        cy = jnp.sum(jnp.where(onehot, Y, 0.0), axis=1, keepdims=True)
        cz = jnp.sum(jnp.where(onehot, Z, 0.0), axis=1, keepdims=True)
        dx = X - cx
        dy = Y - cy
        dz = Z - cz
        d = dx * dx + dy * dy + dz * dz
        dist = jnp.minimum(dist_ref[...], d)
        dist_ref[...] = dist
        m = jnp.max(dist, axis=1, keepdims=True)
        far_new = jnp.min(
            jnp.where(dist == m, iota, N), axis=1, keepdims=True
        ).astype(jnp.int32)
        qx_ref[pl.ds(i, 1), :] = row(cx)
        qy_ref[pl.ds(i, 1), :] = row(cy)
        qz_ref[pl.ds(i, 1), :] = row(cz)
        return far_new

    lax.fori_loop(0, S, body, jnp.zeros((B, 1), jnp.int32))


def _fps(X, Y, Z, S):
    B, N = X.shape
    out = jax.ShapeDtypeStruct((S, B), jnp.float32)
    qx, qy, qz = pl.pallas_call(
        _fps_body,
        out_shape=(out, out, out),
        scratch_shapes=[pltpu.VMEM((B, N), jnp.float32)],
    )(X, Y, Z)
    return qx.T, qy.T, qz.T


def _ballq_body(p_ref, q_ref, o1_ref, o2_ref, d_ref, v_ref, *, N, r1, r2, K1, K2):
    P = p_ref[0]
    Q = q_ref[0]
    Sblk = Q.shape[0]
    pn = jnp.sum(P * P, axis=0, keepdims=True)
    qn = jnp.sum(Q * Q, axis=1, keepdims=True)
    cross = jnp.dot(Q, P, preferred_element_type=jnp.float32)
    d_ref[...] = qn + pn - 2.0 * cross
    iota = lax.broadcasted_iota(jnp.int32, (Sblk, N), 1)
    off = pl.program_id(0) * N

    for r, K, o_ref in ((r1, K1, o1_ref), (r2, K2, o2_ref)):
        vals0 = jnp.where(d_ref[...] > r * r, N, iota)
        maxc = jnp.max(jnp.sum((vals0 < N).astype(jnp.int32), axis=1))
        first = jnp.min(vals0, axis=1, keepdims=True)
        firstc = jnp.minimum(first, N - 1) + off
        o_ref[0] = jnp.broadcast_to(firstc, (Sblk, K)).astype(jnp.int32)
        v_ref[...] = jnp.where(vals0 == first, N, vals0)
        for k in range(1, K):

            @pl.when(k < maxc)
            def _(k=k, o_ref=o_ref, firstc=firstc):
                v = v_ref[...]
                m = jnp.min(v, axis=1, keepdims=True)
                col = jnp.where(m == N, firstc, m + off)
                o_ref[0, :, k : k + 1] = col.astype(jnp.int32)
                v_ref[...] = jnp.where(v == m, N, v)


def _ballq(P, Q, r1, K1, r2, K2, Sblk):
    B, _, N = P.shape
    S = Q.shape[1]
    body = functools.partial(_ballq_body, N=N, r1=r1, r2=r2, K1=K1, K2=K2)
    g1, g2 = pl.pallas_call(
        body,
        grid=(B, S // Sblk),
        in_specs=[
            pl.BlockSpec((1, 8, N), lambda b, s: (b, 0, 0)),
            pl.BlockSpec((1, Sblk, 8), lambda b, s: (b, s, 0)),
        ],
        out_specs=(
            pl.BlockSpec((1, Sblk, K1), lambda b, s: (b, s, 0)),
            pl.BlockSpec((1, Sblk, K2), lambda b, s: (b, s, 0)),
        ),
        out_shape=(
            jax.ShapeDtypeStruct((B, S, K1), jnp.int32),
            jax.ShapeDtypeStruct((B, S, K2), jnp.int32),
        ),
        scratch_shapes=[
            pltpu.VMEM((Sblk, N), jnp.float32),
            pltpu.VMEM((Sblk, N), jnp.int32),
        ],
    )(P, Q)
    return g1, g2


def _atable_body(f_ref, w1_ref, b1_ref, w2_ref, b2_ref, a1_ref, a2_ref):
    F = f_ref[...]
    a1_ref[...] = (
        jnp.dot(F, w1_ref[...], preferred_element_type=jnp.float32) + b1_ref[...]
    )
    a2_ref[...] = (
        jnp.dot(F, w2_ref[...], preferred_element_type=jnp.float32) + b2_ref[...]
    )


def _atable(F, W1, b1, W2, b2, blk):
    R, Cp = F.shape
    C1 = W1.shape[1]
    C2 = W2.shape[1]
    return pl.pallas_call(
        _atable_body,
        grid=(R // blk,),
        in_specs=[
            pl.BlockSpec((blk, Cp), lambda i: (i, 0)),
            pl.BlockSpec((Cp, C1), lambda i: (0, 0)),
            pl.BlockSpec((1, C1), lambda i: (0, 0)),
            pl.BlockSpec((Cp, C2), lambda i: (0, 0)),
            pl.BlockSpec((1, C2), lambda i: (0, 0)),
        ],
        out_specs=(
            pl.BlockSpec((blk, C1), lambda i: (i, 0)),
            pl.BlockSpec((blk, C2), lambda i: (i, 0)),
        ),
        out_shape=(
            jax.ShapeDtypeStruct((R, C1), jnp.float32),
            jax.ShapeDtypeStruct((R, C2), jnp.float32),
        ),
    )(F, W1, b1, W2, b2)


def _sc_gather(table, idx, R, C):
    info = plsc.get_sparse_core_info()
    NW = info.num_cores * info.num_subcores
    per_w = R // NW
    chunk = min(128, per_w)
    n_chunks = per_w // chunk
    G = max(1, min(n_chunks, 96000 // (chunk * C), 16))
    groups = []
    j = 0
    while j < n_chunks:
        groups.append((j, min(G, n_chunks - j)))
        j += G
    mesh = plsc.VectorSubcoreMesh(core_axis_name="c", subcore_axis_name="s")

    @functools.partial(
        pl.kernel,
        mesh=mesh,
        out_type=jax.ShapeDtypeStruct((R, C), jnp.float32),
        scratch_types=[
            pltpu.VMEM((n_chunks, chunk), jnp.int32),
            pltpu.VMEM((G * chunk, C), jnp.float32),
            pltpu.SemaphoreType.DMA,
        ],
    )
    def k(table_hbm, idx_hbm, out_hbm, idx_v, rows_v, sem):
        wid = lax.axis_index("s") * info.num_cores + lax.axis_index("c")
        base = wid * per_w
        pltpu.sync_copy(
            idx_hbm.at[pl.ds(wid * n_chunks, n_chunks), :], idx_v
        )
        for g0, glen in groups:
            copies = []
            for t in range(glen):
                copies.append(
                    pltpu.async_copy(
                        table_hbm.at[idx_v.at[g0 + t]],
                        rows_v.at[pl.ds(t * chunk, chunk), :],
                        sem,
                    )
                )
            for c in copies:
                c.wait()
            pltpu.sync_copy(
                rows_v.at[pl.ds(0, glen * chunk), :],
                out_hbm.at[pl.ds(base + g0 * chunk, glen * chunk), :],
            )

    idx2d = idx.reshape(NW * n_chunks, chunk)
    return k(table, idx2d)


def _mlp_max_body(g_ref, q_ref, wq_ref, w2_ref, b2_ref, w3_ref, b3_ref, o_ref, *, K):
    G = g_ref[0]
    q = q_ref[0]
    Sblk = q.shape[0]
    C1 = G.shape[1]
    bq = jnp.dot(q, wq_ref[...], preferred_element_type=jnp.float32)
    H = jnp.maximum(G.reshape(Sblk, K, C1) - bq[:, None, :], 0.0)
    H = H.reshape(Sblk * K, C1)
    H = jnp.maximum(
        jnp.dot(H, w2_ref[...], preferred_element_type=jnp.float32) + b2_ref[...], 0.0
    )
    H = jnp.maximum(
        jnp.dot(H, w3_ref[...], preferred_element_type=jnp.float32) + b3_ref[...], 0.0
    )
    C3 = H.shape[1]
    o_ref[0] = jnp.max(H.reshape(Sblk, K, C3), axis=1)


def _mlp_max(Gf, Q, Wq, W2, b2, W3, b3, K, Sblk):
    B, SK, C1 = Gf.shape
    S = SK // K
    C2 = W2.shape[1]
    C3 = W3.shape[1]
    body = functools.partial(_mlp_max_body, K=K)
    return pl.pallas_call(
        body,
        grid=(B, S // Sblk),
        in_specs=[
            pl.BlockSpec((1, Sblk * K, C1), lambda b, s: (b, s, 0)),
            pl.BlockSpec((1, Sblk, 8), lambda b, s: (b, s, 0)),
            pl.BlockSpec((8, C1), lambda b, s: (0, 0)),
            pl.BlockSpec((C1, C2), lambda b, s: (0, 0)),
            pl.BlockSpec((1, C2), lambda b, s: (0, 0)),
            pl.BlockSpec((C2, C3), lambda b, s: (0, 0)),
            pl.BlockSpec((1, C3), lambda b, s: (0, 0)),
        ],
        out_specs=pl.BlockSpec((1, Sblk, C3), lambda b, s: (b, s, 0)),
        out_shape=jax.ShapeDtypeStruct((B, S, C3), jnp.float32),
    )(Gf, Q, Wq, W2, b2, W3, b3)


def _fp_body(x1_ref, x2_ref, p2_ref, *rest, n_layers, has_p1, has_cls, S):
    i = 0
    p1_ref = None
    if has_p1:
        p1_ref = rest[0]
        i = 1
    w_int = rest[i]
    w_p1 = rest[i + 1] if has_p1 else None
    b_1 = rest[i + 1 + (1 if has_p1 else 0)]
    rest = rest[i + 2 + (1 if has_p1 else 0) :]
    layer_ws = []
    for _ in range(n_layers - 1):
        layer_ws.append((rest[0], rest[1]))
        rest = rest[2:]
    cls_ws = None
    if has_cls:
        cls_ws = (rest[0], rest[1], rest[2], rest[3])
        rest = rest[4:]
    o_ref = rest[0]

    q = x1_ref[0]
    p = x2_ref[0]
    Nblk = q.shape[0]
    qn = jnp.sum(q * q, axis=1, keepdims=True)
    psq = p * p
    pn = lax.dot_general(
        jnp.ones((1, 8), jnp.float32),
        psq,
        (((1,), (1,)), ((), ())),
        preferred_element_type=jnp.float32,
    )
    cross = lax.dot_general(
        q, p, (((1,), (1,)), ((), ())), preferred_element_type=jnp.float32
    )
    d = qn + pn - 2.0 * cross
    iota = lax.broadcasted_iota(jnp.int32, (Nblk, S), 1)
    vals = d
    rs = []
    ohs = []
    for _ in range(3):
        m = jnp.min(vals, axis=1, keepdims=True)
        isel = jnp.min(jnp.where(vals == m, iota, S), axis=1, keepdims=True)
        oh = iota == isel
        rs.append(1.0 / (m + 1e-8))
        ohs.append(oh)
        vals = jnp.where(oh, 1e30, vals)
    tot = rs[0] + rs[1] + rs[2]
    Wm = (
        jnp.where(ohs[0], rs[0] / tot, 0.0)
        + jnp.where(ohs[1], rs[1] / tot, 0.0)
        + jnp.where(ohs[2], rs[2] / tot, 0.0)
    )
    interp = jnp.dot(Wm, p2_ref[0], preferred_element_type=jnp.float32)
    acc = jnp.dot(interp, w_int[...], preferred_element_type=jnp.float32)
    if has_p1:
        acc = acc + jnp.dot(
            p1_ref[0], w_p1[...], preferred_element_type=jnp.float32
        )
    H = jnp.maximum(acc + b_1[...], 0.0)
    for w, b in layer_ws:
        H = jnp.maximum(
            jnp.dot(H, w[...], preferred_element_type=jnp.float32) + b[...], 0.0
        )
    if has_cls:
        wc1, bc1, wc2, bc2 = cls_ws
        H = jnp.maximum(
            jnp.dot(H, wc1[...], preferred_element_type=jnp.float32) + bc1[...], 0.0
        )
        logits = jnp.dot(H, wc2[...], preferred_element_type=jnp.float32) + bc2[...]
        mx = jnp.max(logits, axis=1, keepdims=True)
        lse = jnp.log(jnp.sum(jnp.exp(logits - mx), axis=1, keepdims=True)) + mx
        o_ref[0] = logits - lse
    else:
        o_ref[0] = H


def _fp(X1, X2, P2, P1, layers, cls, Nblk):
    B, N, _ = X1.shape
    S = X2.shape[1]
    C2 = P2.shape[2]
    has_p1 = P1 is not None
    C1 = P1.shape[2] if has_p1 else 0

    (W1, b1) = layers[0]
    w_int = W1[C1:, :]
    w_p1 = W1[:C1, :] if has_p1 else None

    args = [X1, X2, P2]
    specs = [
        pl.BlockSpec((1, Nblk, 8), lambda b, n: (b, n, 0)),
        pl.BlockSpec((1, S, 8), lambda b, n: (b, 0, 0)),
        pl.BlockSpec((1, S, C2), lambda b, n: (b, 0, 0)),
    ]
    if has_p1:
        args.append(P1)
        specs.append(pl.BlockSpec((1, Nblk, C1), lambda b, n: (b, n, 0)))
    args.append(w_int)
    specs.append(pl.BlockSpec(w_int.shape, lambda b, n: (0, 0)))
    if has_p1:
        args.append(w_p1)
        specs.append(pl.BlockSpec(w_p1.shape, lambda b, n: (0, 0)))
    args.append(b1.reshape(1, -1))
    specs.append(pl.BlockSpec((1, b1.shape[0]), lambda b, n: (0, 0)))
    for W, b in layers[1:]:
        args.append(W)
        specs.append(pl.BlockSpec(W.shape, lambda b, n: (0, 0)))
        args.append(b.reshape(1, -1))
        specs.append(pl.BlockSpec((1, b.shape[0]), lambda b, n: (0, 0)))
    if cls is not None:
        for arr in cls:
            args.append(arr)
            specs.append(pl.BlockSpec(arr.shape, lambda b, n: (0, 0)))
    Cout = 13 if cls is not None else layers[-1][0].shape[1]
    body = functools.partial(
        _fp_body,
        n_layers=len(layers),
        has_p1=has_p1,
        has_cls=cls is not None,
        S=S,
    )
    return pl.pallas_call(
        body,
        grid=(B, N // Nblk),
        in_specs=specs,
        out_specs=pl.BlockSpec((1, Nblk, Cout), lambda b, n: (b, n, 0)),
        out_shape=jax.ShapeDtypeStruct((B, N, Cout), jnp.float32),
    )(*args)


def _pad_lanes(x, mult):
    c = x.shape[-1]
    pad = (-c) % mult
    if pad == 0:
        return x
    return jnp.concatenate(
        [x, jnp.zeros(x.shape[:-1] + (pad,), x.dtype)], axis=-1
    )


def _coords_rows(X, Y, Z):
    B, N = X.shape
    return jnp.concatenate(
        [X[:, None], Y[:, None], Z[:, None], jnp.zeros((B, 5, N), X.dtype)], axis=1
    )


def _coords_cols(X, Y, Z):
    B, N = X.shape
    return jnp.concatenate(
        [X[..., None], Y[..., None], Z[..., None], jnp.zeros((B, N, 5), X.dtype)],
        axis=-1,
    )


def _sa_level(X, Y, Z, pts, cfg, branches, Sblk_bq, Sblk_mlp):
    S, (r1, r2), (K1, K2) = cfg
    B, N = X.shape
    Cin = pts.shape[2]

    QX, QY, QZ = _fps(X, Y, Z, S)

    xyz_cols = jnp.stack([X, Y, Z], axis=-1)
    F = jnp.concatenate([pts, xyz_cols], axis=-1)
    Cp = F.shape[-1]
    Fp = _pad_lanes(F, 128).reshape(B * N, -1)
    Cpad = Fp.shape[1]

    folded = [[_fold(p) for p in br] for br in branches]
    (W1a, b1a), (W2a, b2a), (W3a, b3a) = folded[0]
    (W1b, b1b), (W2b, b2b), (W3b, b3b) = folded[1]
    C1a = -(-W1a.shape[0] // 128) * 128
    C1b = -(-W1b.shape[0] // 128) * 128
    W1at = (
        jnp.zeros((Cpad, C1a), jnp.float32).at[:Cp, : W1a.shape[0]].set(W1a.T)
    )
    W1bt = (
        jnp.zeros((Cpad, C1b), jnp.float32).at[:Cp, : W1b.shape[0]].set(W1b.T)
    )
    b1a_p = jnp.zeros((1, C1a), jnp.float32).at[:, : W1a.shape[0]].set(b1a)
    b1b_p = jnp.zeros((1, C1b), jnp.float32).at[:, : W1b.shape[0]].set(b1b)
    A1, A2 = _atable(Fp, W1at, b1a_p, W1bt, b1b_p, blk=min(512, B * N))

    P = _coords_rows(X, Y, Z)
    Q = _coords_cols(QX, QY, QZ)
    g1, g2 = _ballq(P, Q, r1, K1, r2, K2, Sblk_bq)

    G1 = _sc_gather(A1, g1.reshape(-1), B * S * K1, C1a).reshape(B, S * K1, C1a)
    G2 = _sc_gather(A2, g2.reshape(-1), B * S * K2, C1b).reshape(B, S * K2, C1b)

    def wq(W1, C1p):
        return (
            jnp.zeros((8, C1p), jnp.float32)
            .at[:3, : W1.shape[0]]
            .set(W1[:, Cin : Cin + 3].T)
        )

    def wpad(W2, C1p):
        return jnp.zeros((C1p, W2.shape[0]), jnp.float32).at[: W2.shape[1]].set(
            W2.T
        )

    o1 = _mlp_max(
        G1, Q, wq(W1a, C1a), wpad(W2a, C1a), b2a.reshape(1, -1),
        W3a.T, b3a.reshape(1, -1), K1, min(Sblk_mlp, S),
    )
    o2 = _mlp_max(
        G2, Q, wq(W1b, C1b), wpad(W2b, C1b), b2b.reshape(1, -1),
        W3b.T, b3b.reshape(1, -1), K2, min(Sblk_mlp, S),
    )
    return QX, QY, QZ, jnp.concatenate([o1, o2], axis=-1)


def kernel(point_cloud, params):
    pc = point_cloud
    B, C0, N0 = pc.shape
    X0 = pc[:, 0, :]
    Y0 = pc[:, 1, :]
    Z0 = pc[:, 2, :]
    pts0 = jnp.transpose(pc, (0, 2, 1))

    QX1, QY1, QZ1, p1 = _sa_level(
        X0, Y0, Z0, pts0, _SA_CFGS[0], params["sa1"], Sblk_bq=128, Sblk_mlp=64
    )
    QX2, QY2, QZ2, p2 = _sa_level(
        QX1, QY1, QZ1, p1, _SA_CFGS[1], params["sa2"], Sblk_bq=128, Sblk_mlp=64
    )
    QX3, QY3, QZ3, p3 = _sa_level(
        QX2, QY2, QZ2, p2, _SA_CFGS[2], params["sa3"], Sblk_bq=64, Sblk_mlp=64
    )
    QX4, QY4, QZ4, p4 = _sa_level(
        QX3, QY3, QZ3, p3, _SA_CFGS[3], params["sa4"], Sblk_bq=16, Sblk_mlp=16
    )

    x1c = _coords_cols(QX1, QY1, QZ1)
    x2c = _coords_cols(QX2, QY2, QZ2)
    x3c = _coords_cols(QX3, QY3, QZ3)
    x4c = _coords_cols(QX4, QY4, QZ4)
    x0c = _coords_cols(X0, Y0, Z0)

    def fold_layers(ps):
        out = []
        for p in ps:
            W, b = _fold(p)
            out.append((W.T, b))
        return out

    fp1l = fold_layers(params["fp1"])
    fp2l = fold_layers(params["fp2"])
    fp3l = fold_layers(params["fp3"])
    fp4l = fold_layers(params["fp4"])
    Wc1, bc1 = _fold(params["cls1"])
    cls = (
        Wc1.T,
        bc1.reshape(1, -1),
        params["cls2"]["W"].T,
        params["cls2"]["b"].reshape(1, -1),
    )

    u1 = _fp(x3c, x4c, p4, p3, fp1l, None, Nblk=64)
    u2 = _fp(x2c, x3c, u1, p2, fp2l, None, Nblk=128)
    u3 = _fp(x1c, x2c, u2, p1, fp3l, None, Nblk=256)
    pred = _fp(x0c, x1c, u3, None, fp4l, cls, Nblk=256)

    p4_out = jnp.transpose(p4, (0, 2, 1))
    return pred, p4_out


# speedup vs baseline: 26.5530x; 1.2978x over previous
import functools

import jax
import jax.numpy as jnp
import numpy as np
from jax import lax
from jax.experimental import pallas as pl
from jax.experimental.pallas import tpu as pltpu
from jax.experimental.pallas import tpu_sc as plsc

_SA_CFGS = [
    (1024, [0.05, 0.1], [16, 32]),
    (256, [0.1, 0.2], [16, 32]),
    (64, [0.2, 0.4], [16, 32]),
    (16, [0.4, 0.8], [16, 32]),
]
_B = 8
_N0 = 4096


def _fold(p):
    s = p["gamma"] / np.sqrt(1.0 + 1e-5)
    return p["W"] * s[:, None], p["b"] * s + p["beta"]


def _fps_body(x_ref, y_ref, z_ref, qx_ref, qy_ref, qz_ref, dist_ref):
    B, N = x_ref.shape
    S = qx_ref.shape[0]
    X = x_ref[...]
    Y = y_ref[...]
    Z = z_ref[...]
    iota = lax.broadcasted_iota(jnp.int32, (B, N), 1)
    eye = lax.broadcasted_iota(jnp.int32, (B, B), 0) == lax.broadcasted_iota(
        jnp.int32, (B, B), 1
    )
    dist_ref[...] = jnp.full((B, N), 1e10, jnp.float32)

    def row(v):
        return jnp.sum(jnp.where(eye, v, 0.0), axis=0, keepdims=True)

    def body(i, far):
        onehot = iota == far
        cx = jnp.sum(jnp.where(onehot, X, 0.0), axis=1, keepdims=True)
        cy = jnp.sum(jnp.where(onehot, Y, 0.0), axis=1, keepdims=True)
        cz = jnp.sum(jnp.where(onehot, Z, 0.0), axis=1, keepdims=True)
        dx = X - cx
        dy = Y - cy
        dz = Z - cz
        d = dx * dx + dy * dy + dz * dz
        dist = jnp.minimum(dist_ref[...], d)
        dist_ref[...] = dist
        m = jnp.max(dist, axis=1, keepdims=True)
        far_new = jnp.min(
            jnp.where(dist == m, iota, N), axis=1, keepdims=True
        ).astype(jnp.int32)
        qx_ref[pl.ds(i, 1), :] = row(cx)
        qy_ref[pl.ds(i, 1), :] = row(cy)
        qz_ref[pl.ds(i, 1), :] = row(cz)
        return far_new

    lax.fori_loop(0, S, body, jnp.zeros((B, 1), jnp.int32))


def _fps(X, Y, Z, S):
    B, N = X.shape
    out = jax.ShapeDtypeStruct((S, B), jnp.float32)
    qx, qy, qz = pl.pallas_call(
        _fps_body,
        out_shape=(out, out, out),
        scratch_shapes=[pltpu.VMEM((B, N), jnp.float32)],
    )(X, Y, Z)
    return qx.T, qy.T, qz.T


def _ballq_body(p_ref, q_ref, o1_ref, o2_ref, d_ref, v_ref, *, N, r1, r2, K1, K2):
    P = p_ref[0]
    Q = q_ref[0]
    Sblk = Q.shape[0]
    pn = jnp.sum(P * P, axis=0, keepdims=True)
    qn = jnp.sum(Q * Q, axis=1, keepdims=True)
    cross = jnp.dot(Q, P, preferred_element_type=jnp.float32)
    d_ref[...] = qn + pn - 2.0 * cross
    iota = lax.broadcasted_iota(jnp.int32, (Sblk, N), 1)
    off = pl.program_id(0) * N

    for r, K, o_ref in ((r1, K1, o1_ref), (r2, K2, o2_ref)):
        vals0 = jnp.where(d_ref[...] > r * r, N, iota)
        maxc = jnp.max(jnp.sum((vals0 < N).astype(jnp.int32), axis=1))
        first = jnp.min(vals0, axis=1, keepdims=True)
        firstc = jnp.minimum(first, N - 1) + off
        o_ref[0] = jnp.broadcast_to(firstc, (Sblk, K)).astype(jnp.int32)
        v_ref[...] = jnp.where(vals0 == first, N, vals0)
        for k in range(1, K):

            @pl.when(k < maxc)
            def _(k=k, o_ref=o_ref, firstc=firstc):
                v = v_ref[...]
                m = jnp.min(v, axis=1, keepdims=True)
                col = jnp.where(m == N, firstc, m + off)
                o_ref[0, :, k : k + 1] = col.astype(jnp.int32)
                v_ref[...] = jnp.where(v == m, N, v)


def _ballq(P, Q, r1, K1, r2, K2, Sblk):
    B, _, N = P.shape
    S = Q.shape[1]
    body = functools.partial(_ballq_body, N=N, r1=r1, r2=r2, K1=K1, K2=K2)
    g1, g2 = pl.pallas_call(
        body,
        grid=(B, S // Sblk),
        in_specs=[
            pl.BlockSpec((1, 8, N), lambda b, s: (b, 0, 0)),
            pl.BlockSpec((1, Sblk, 8), lambda b, s: (b, s, 0)),
        ],
        out_specs=(
            pl.BlockSpec((1, Sblk, K1), lambda b, s: (b, s, 0)),
            pl.BlockSpec((1, Sblk, K2), lambda b, s: (b, s, 0)),
        ),
        out_shape=(
            jax.ShapeDtypeStruct((B, S, K1), jnp.int32),
            jax.ShapeDtypeStruct((B, S, K2), jnp.int32),
        ),
        scratch_shapes=[
            pltpu.VMEM((Sblk, N), jnp.float32),
            pltpu.VMEM((Sblk, N), jnp.int32),
        ],
    )(P, Q)
    return g1, g2


def _atable_body(f_ref, w1_ref, b1_ref, w2_ref, b2_ref, a1_ref, a2_ref):
    F = f_ref[...]
    a1_ref[...] = (
        jnp.dot(F, w1_ref[...], preferred_element_type=jnp.float32) + b1_ref[...]
    )
    a2_ref[...] = (
        jnp.dot(F, w2_ref[...], preferred_element_type=jnp.float32) + b2_ref[...]
    )


def _atable(F, W1, b1, W2, b2, blk):
    R, Cp = F.shape
    C1 = W1.shape[1]
    C2 = W2.shape[1]
    return pl.pallas_call(
        _atable_body,
        grid=(R // blk,),
        in_specs=[
            pl.BlockSpec((blk, Cp), lambda i: (i, 0)),
            pl.BlockSpec((Cp, C1), lambda i: (0, 0)),
            pl.BlockSpec((1, C1), lambda i: (0, 0)),
            pl.BlockSpec((Cp, C2), lambda i: (0, 0)),
            pl.BlockSpec((1, C2), lambda i: (0, 0)),
        ],
        out_specs=(
            pl.BlockSpec((blk, C1), lambda i: (i, 0)),
            pl.BlockSpec((blk, C2), lambda i: (i, 0)),
        ),
        out_shape=(
            jax.ShapeDtypeStruct((R, C1), jnp.float32),
            jax.ShapeDtypeStruct((R, C2), jnp.float32),
        ),
    )(F, W1, b1, W2, b2)


def _sc_gather(table, idx, R, C):
    info = plsc.get_sparse_core_info()
    NW = info.num_cores * info.num_subcores
    per_w = R // NW
    chunk = min(128, per_w)
    n_chunks = per_w // chunk
    G = max(1, min(n_chunks, 96000 // (chunk * C), 16))
    groups = []
    j = 0
    while j < n_chunks:
        groups.append((j, min(G, n_chunks - j)))
        j += G
    mesh = plsc.VectorSubcoreMesh(core_axis_name="c", subcore_axis_name="s")

    @functools.partial(
        pl.kernel,
        mesh=mesh,
        out_type=jax.ShapeDtypeStruct((R, C), jnp.float32),
        compiler_params=pltpu.CompilerParams(use_tc_tiling_on_sc=False),
        scratch_types=[
            pltpu.VMEM((n_chunks, chunk), jnp.int32),
            pltpu.VMEM((G * chunk, C), jnp.float32),
            pltpu.SemaphoreType.DMA,
        ],
    )
    def k(table_hbm, idx_hbm, out_hbm, idx_v, rows_v, sem):
        wid = lax.axis_index("s") * info.num_cores + lax.axis_index("c")
        base = wid * per_w
        pltpu.sync_copy(
            idx_hbm.at[pl.ds(wid * n_chunks, n_chunks), :], idx_v
        )
        for g0, glen in groups:
            copies = []
            for t in range(glen):
                copies.append(
                    pltpu.async_copy(
                        table_hbm.at[idx_v.at[g0 + t]],
                        rows_v.at[pl.ds(t * chunk, chunk), :],
                        sem,
                    )
                )
            for c in copies:
                c.wait()
            pltpu.sync_copy(
                rows_v.at[pl.ds(0, glen * chunk), :],
                out_hbm.at[pl.ds(base + g0 * chunk, glen * chunk), :],
            )

    idx2d = idx.reshape(NW * n_chunks, chunk)
    return k(table, idx2d)


def _mlp_max_body(g_ref, q_ref, wq_ref, w2_ref, b2_ref, w3_ref, b3_ref, o_ref, *, K):
    G = g_ref[0]
    q = q_ref[0]
    Sblk = q.shape[0]
    C1 = G.shape[1]
    bq = jnp.dot(q, wq_ref[...], preferred_element_type=jnp.float32)
    H = jnp.maximum(G.reshape(Sblk, K, C1) - bq[:, None, :], 0.0)
    H = H.reshape(Sblk * K, C1)
    H = jnp.maximum(
        jnp.dot(H, w2_ref[...], preferred_element_type=jnp.float32) + b2_ref[...], 0.0
    )
    H = jnp.maximum(
        jnp.dot(H, w3_ref[...], preferred_element_type=jnp.float32) + b3_ref[...], 0.0
    )
    C3 = H.shape[1]
    o_ref[0] = jnp.max(H.reshape(Sblk, K, C3), axis=1)


def _mlp_max(Gf, Q, Wq, W2, b2, W3, b3, K, Sblk):
    B, SK, C1 = Gf.shape
    S = SK // K
    C2 = W2.shape[1]
    C3 = W3.shape[1]
    body = functools.partial(_mlp_max_body, K=K)
    return pl.pallas_call(
        body,
        grid=(B, S // Sblk),
        in_specs=[
            pl.BlockSpec((1, Sblk * K, C1), lambda b, s: (b, s, 0)),
            pl.BlockSpec((1, Sblk, 8), lambda b, s: (b, s, 0)),
            pl.BlockSpec((8, C1), lambda b, s: (0, 0)),
            pl.BlockSpec((C1, C2), lambda b, s: (0, 0)),
            pl.BlockSpec((1, C2), lambda b, s: (0, 0)),
            pl.BlockSpec((C2, C3), lambda b, s: (0, 0)),
            pl.BlockSpec((1, C3), lambda b, s: (0, 0)),
        ],
        out_specs=pl.BlockSpec((1, Sblk, C3), lambda b, s: (b, s, 0)),
        out_shape=jax.ShapeDtypeStruct((B, S, C3), jnp.float32),
    )(Gf, Q, Wq, W2, b2, W3, b3)


def _fp_body(x1_ref, x2_ref, p2_ref, *rest, n_layers, has_p1, has_cls, S):
    i = 0
    p1_ref = None
    if has_p1:
        p1_ref = rest[0]
        i = 1
    w_int = rest[i]
    w_p1 = rest[i + 1] if has_p1 else None
    b_1 = rest[i + 1 + (1 if has_p1 else 0)]
    rest = rest[i + 2 + (1 if has_p1 else 0) :]
    layer_ws = []
    for _ in range(n_layers - 1):
        layer_ws.append((rest[0], rest[1]))
        rest = rest[2:]
    cls_ws = None
    if has_cls:
        cls_ws = (rest[0], rest[1], rest[2], rest[3])
        rest = rest[4:]
    o_ref = rest[0]

    q = x1_ref[0]
    p = x2_ref[0]
    Nblk = q.shape[0]
    qn = jnp.sum(q * q, axis=1, keepdims=True)
    psq = p * p
    pn = lax.dot_general(
        jnp.ones((1, 8), jnp.float32),
        psq,
        (((1,), (1,)), ((), ())),
        preferred_element_type=jnp.float32,
    )
    cross = lax.dot_general(
        q, p, (((1,), (1,)), ((), ())), preferred_element_type=jnp.float32
    )
    d = qn + pn - 2.0 * cross
    iota = lax.broadcasted_iota(jnp.int32, (Nblk, S), 1)
    vals = d
    rs = []
    ohs = []
    for _ in range(3):
        m = jnp.min(vals, axis=1, keepdims=True)
        isel = jnp.min(jnp.where(vals == m, iota, S), axis=1, keepdims=True)
        oh = iota == isel
        rs.append(1.0 / (m + 1e-8))
        ohs.append(oh)
        vals = jnp.where(oh, 1e30, vals)
    tot = rs[0] + rs[1] + rs[2]
    Wm = (
        jnp.where(ohs[0], rs[0] / tot, 0.0)
        + jnp.where(ohs[1], rs[1] / tot, 0.0)
        + jnp.where(ohs[2], rs[2] / tot, 0.0)
    )
    interp = jnp.dot(Wm, p2_ref[0], preferred_element_type=jnp.float32)
    acc = jnp.dot(interp, w_int[...], preferred_element_type=jnp.float32)
    if has_p1:
        acc = acc + jnp.dot(
            p1_ref[0], w_p1[...], preferred_element_type=jnp.float32
        )
    H = jnp.maximum(acc + b_1[...], 0.0)
    for w, b in layer_ws:
        H = jnp.maximum(
            jnp.dot(H, w[...], preferred_element_type=jnp.float32) + b[...], 0.0
        )
    if has_cls:
        wc1, bc1, wc2, bc2 = cls_ws
        H = jnp.maximum(
            jnp.dot(H, wc1[...], preferred_element_type=jnp.float32) + bc1[...], 0.0
        )
        logits = jnp.dot(H, wc2[...], preferred_element_type=jnp.float32) + bc2[...]
        mx = jnp.max(logits, axis=1, keepdims=True)
        lse = jnp.log(jnp.sum(jnp.exp(logits - mx), axis=1, keepdims=True)) + mx
        o_ref[0] = logits - lse
    else:
        o_ref[0] = H


def _fp(X1, X2, P2, P1, layers, cls, Nblk):
    B, N, _ = X1.shape
    S = X2.shape[1]
    C2 = P2.shape[2]
    has_p1 = P1 is not None
    C1 = P1.shape[2] if has_p1 else 0

    (W1, b1) = layers[0]
    w_int = W1[C1:, :]
    w_p1 = W1[:C1, :] if has_p1 else None

    args = [X1, X2, P2]
    specs = [
        pl.BlockSpec((1, Nblk, 8), lambda b, n: (b, n, 0)),
        pl.BlockSpec((1, S, 8), lambda b, n: (b, 0, 0)),
        pl.BlockSpec((1, S, C2), lambda b, n: (b, 0, 0)),
    ]
    if has_p1:
        args.append(P1)
        specs.append(pl.BlockSpec((1, Nblk, C1), lambda b, n: (b, n, 0)))
    args.append(w_int)
    specs.append(pl.BlockSpec(w_int.shape, lambda b, n: (0, 0)))
    if has_p1:
        args.append(w_p1)
        specs.append(pl.BlockSpec(w_p1.shape, lambda b, n: (0, 0)))
    args.append(b1.reshape(1, -1))
    specs.append(pl.BlockSpec((1, b1.shape[0]), lambda b, n: (0, 0)))
    for W, b in layers[1:]:
        args.append(W)
        specs.append(pl.BlockSpec(W.shape, lambda b, n: (0, 0)))
        args.append(b.reshape(1, -1))
        specs.append(pl.BlockSpec((1, b.shape[0]), lambda b, n: (0, 0)))
    if cls is not None:
        for arr in cls:
            args.append(arr)
            specs.append(pl.BlockSpec(arr.shape, lambda b, n: (0, 0)))
    Cout = 13 if cls is not None else layers[-1][0].shape[1]
    body = functools.partial(
        _fp_body,
        n_layers=len(layers),
        has_p1=has_p1,
        has_cls=cls is not None,
        S=S,
    )
    return pl.pallas_call(
        body,
        grid=(B, N // Nblk),
        in_specs=specs,
        out_specs=pl.BlockSpec((1, Nblk, Cout), lambda b, n: (b, n, 0)),
        out_shape=jax.ShapeDtypeStruct((B, N, Cout), jnp.float32),
    )(*args)


def _pad_lanes(x, mult):
    c = x.shape[-1]
    pad = (-c) % mult
    if pad == 0:
        return x
    return jnp.concatenate(
        [x, jnp.zeros(x.shape[:-1] + (pad,), x.dtype)], axis=-1
    )


def _coords_rows(X, Y, Z):
    B, N = X.shape
    return jnp.concatenate(
        [X[:, None], Y[:, None], Z[:, None], jnp.zeros((B, 5, N), X.dtype)], axis=1
    )


def _coords_cols(X, Y, Z):
    B, N = X.shape
    return jnp.concatenate(
        [X[..., None], Y[..., None], Z[..., None], jnp.zeros((B, N, 5), X.dtype)],
        axis=-1,
    )


def _sa_level(X, Y, Z, pts, cfg, branches, Sblk_bq, Sblk_mlp):
    S, (r1, r2), (K1, K2) = cfg
    B, N = X.shape
    Cin = pts.shape[2]

    QX, QY, QZ = _fps(X, Y, Z, S)

    xyz_cols = jnp.stack([X, Y, Z], axis=-1)
    F = jnp.concatenate([pts, xyz_cols], axis=-1)
    Cp = F.shape[-1]
    Fp = _pad_lanes(F, 128).reshape(B * N, -1)
    Cpad = Fp.shape[1]

    folded = [[_fold(p) for p in br] for br in branches]
    (W1a, b1a), (W2a, b2a), (W3a, b3a) = folded[0]
    (W1b, b1b), (W2b, b2b), (W3b, b3b) = folded[1]
    C1a = W1a.shape[0]
    C1b = W1b.shape[0]
    W1at = (
        jnp.zeros((Cpad, C1a), jnp.float32).at[:Cp, : W1a.shape[0]].set(W1a.T)
    )
    W1bt = (
        jnp.zeros((Cpad, C1b), jnp.float32).at[:Cp, : W1b.shape[0]].set(W1b.T)
    )
    b1a_p = jnp.zeros((1, C1a), jnp.float32).at[:, : W1a.shape[0]].set(b1a)
    b1b_p = jnp.zeros((1, C1b), jnp.float32).at[:, : W1b.shape[0]].set(b1b)
    A1, A2 = _atable(Fp, W1at, b1a_p, W1bt, b1b_p, blk=min(512, B * N))

    P = _coords_rows(X, Y, Z)
    Q = _coords_cols(QX, QY, QZ)
    g1, g2 = _ballq(P, Q, r1, K1, r2, K2, Sblk_bq)

    G1 = _sc_gather(A1, g1.reshape(-1), B * S * K1, C1a).reshape(B, S * K1, C1a)
    G2 = _sc_gather(A2, g2.reshape(-1), B * S * K2, C1b).reshape(B, S * K2, C1b)

    def wq(W1, C1p):
        return (
            jnp.zeros((8, C1p), jnp.float32)
            .at[:3, : W1.shape[0]]
            .set(W1[:, Cin : Cin + 3].T)
        )

    def wpad(W2, C1p):
        return jnp.zeros((C1p, W2.shape[0]), jnp.float32).at[: W2.shape[1]].set(
            W2.T
        )

    o1 = _mlp_max(
        G1, Q, wq(W1a, C1a), wpad(W2a, C1a), b2a.reshape(1, -1),
        W3a.T, b3a.reshape(1, -1), K1, min(Sblk_mlp, S),
    )
    o2 = _mlp_max(
        G2, Q, wq(W1b, C1b), wpad(W2b, C1b), b2b.reshape(1, -1),
        W3b.T, b3b.reshape(1, -1), K2, min(Sblk_mlp, S),
    )
    return QX, QY, QZ, jnp.concatenate([o1, o2], axis=-1)


def kernel(point_cloud, params):
    pc = point_cloud
    B, C0, N0 = pc.shape
    X0 = pc[:, 0, :]
    Y0 = pc[:, 1, :]
    Z0 = pc[:, 2, :]
    pts0 = jnp.transpose(pc, (0, 2, 1))

    QX1, QY1, QZ1, p1 = _sa_level(
        X0, Y0, Z0, pts0, _SA_CFGS[0], params["sa1"], Sblk_bq=128, Sblk_mlp=64
    )
    QX2, QY2, QZ2, p2 = _sa_level(
        QX1, QY1, QZ1, p1, _SA_CFGS[1], params["sa2"], Sblk_bq=128, Sblk_mlp=64
    )
    QX3, QY3, QZ3, p3 = _sa_level(
        QX2, QY2, QZ2, p2, _SA_CFGS[2], params["sa3"], Sblk_bq=64, Sblk_mlp=64
    )
    QX4, QY4, QZ4, p4 = _sa_level(
        QX3, QY3, QZ3, p3, _SA_CFGS[3], params["sa4"], Sblk_bq=16, Sblk_mlp=16
    )

    x1c = _coords_cols(QX1, QY1, QZ1)
    x2c = _coords_cols(QX2, QY2, QZ2)
    x3c = _coords_cols(QX3, QY3, QZ3)
    x4c = _coords_cols(QX4, QY4, QZ4)
    x0c = _coords_cols(X0, Y0, Z0)

    def fold_layers(ps):
        out = []
        for p in ps:
            W, b = _fold(p)
            out.append((W.T, b))
        return out

    fp1l = fold_layers(params["fp1"])
    fp2l = fold_layers(params["fp2"])
    fp3l = fold_layers(params["fp3"])
    fp4l = fold_layers(params["fp4"])
    Wc1, bc1 = _fold(params["cls1"])
    cls = (
        Wc1.T,
        bc1.reshape(1, -1),
        params["cls2"]["W"].T,
        params["cls2"]["b"].reshape(1, -1),
    )

    u1 = _fp(x3c, x4c, p4, p3, fp1l, None, Nblk=64)
    u2 = _fp(x2c, x3c, u1, p2, fp2l, None, Nblk=128)
    u3 = _fp(x1c, x2c, u2, p1, fp3l, None, Nblk=256)
    pred = _fp(x0c, x1c, u3, None, fp4l, cls, Nblk=256)

    p4_out = jnp.transpose(p4, (0, 2, 1))
    return pred, p4_out


# speedup vs baseline: 27.6893x; 1.0428x over previous
import functools

import jax
import jax.numpy as jnp
import numpy as np
from jax import lax
from jax.experimental import pallas as pl
from jax.experimental.pallas import tpu as pltpu
from jax.experimental.pallas import tpu_sc as plsc

_SA_CFGS = [
    (1024, [0.05, 0.1], [16, 32]),
    (256, [0.1, 0.2], [16, 32]),
    (64, [0.2, 0.4], [16, 32]),
    (16, [0.4, 0.8], [16, 32]),
]
_B = 8
_N0 = 4096


def _fold(p):
    s = p["gamma"] / np.sqrt(1.0 + 1e-5)
    return p["W"] * s[:, None], p["b"] * s + p["beta"]


def _fps_body(x_ref, y_ref, z_ref, qx_ref, qy_ref, qz_ref, dist_ref):
    B, N = x_ref.shape
    S = qx_ref.shape[0]
    X = x_ref[...]
    Y = y_ref[...]
    Z = z_ref[...]
    iota = lax.broadcasted_iota(jnp.int32, (B, N), 1)
    eye = lax.broadcasted_iota(jnp.int32, (B, B), 0) == lax.broadcasted_iota(
        jnp.int32, (B, B), 1
    )
    dist_ref[...] = jnp.full((B, N), 1e10, jnp.float32)

    def row(v):
        return jnp.sum(jnp.where(eye, v, 0.0), axis=0, keepdims=True)

    def body(i, far):
        onehot = iota == far
        cx = jnp.sum(jnp.where(onehot, X, 0.0), axis=1, keepdims=True)
        cy = jnp.sum(jnp.where(onehot, Y, 0.0), axis=1, keepdims=True)
        cz = jnp.sum(jnp.where(onehot, Z, 0.0), axis=1, keepdims=True)
        dx = X - cx
        dy = Y - cy
        dz = Z - cz
        d = dx * dx + dy * dy + dz * dz
        dist = jnp.minimum(dist_ref[...], d)
        dist_ref[...] = dist
        m = jnp.max(dist, axis=1, keepdims=True)
        far_new = jnp.min(
            jnp.where(dist == m, iota, N), axis=1, keepdims=True
        ).astype(jnp.int32)
        qx_ref[pl.ds(i, 1), :] = row(cx)
        qy_ref[pl.ds(i, 1), :] = row(cy)
        qz_ref[pl.ds(i, 1), :] = row(cz)
        return far_new

    lax.fori_loop(0, S, body, jnp.zeros((B, 1), jnp.int32))


def _fps(X, Y, Z, S):
    B, N = X.shape
    out = jax.ShapeDtypeStruct((S, B), jnp.float32)
    qx, qy, qz = pl.pallas_call(
        _fps_body,
        out_shape=(out, out, out),
        scratch_shapes=[pltpu.VMEM((B, N), jnp.float32)],
    )(X, Y, Z)
    return qx.T, qy.T, qz.T


def _ballq_body(
    p_ref, q_ref, o1_ref, o2_ref, v1_ref, v2_ref, m1_ref, m2_ref, *, N, r1, r2, K1, K2
):
    P = p_ref[0]
    Q = q_ref[0]
    Sblk = Q.shape[0]
    pn = jnp.sum(P * P, axis=0, keepdims=True)
    qn = jnp.sum(Q * Q, axis=1, keepdims=True)
    cross = jnp.dot(Q, P, preferred_element_type=jnp.float32)
    d = qn + pn - 2.0 * cross
    iota32 = lax.broadcasted_iota(jnp.int32, (Sblk, N), 1)
    n32 = jnp.int32(N)
    v1_ref[...] = jnp.where(d <= r1 * r1, iota32, n32)
    v2_ref[...] = jnp.where(d <= r2 * r2, iota32, n32)
    off = pl.program_id(0) * N

    for K, v_ref, m_ref, o_ref in (
        (K1, v1_ref, m1_ref, o1_ref),
        (K2, v2_ref, m2_ref, o2_ref),
    ):
        v0 = v_ref[...]
        maxc = jnp.max(jnp.sum((v0 < n32).astype(jnp.int32), axis=1))
        first = jnp.min(v0, axis=1, keepdims=True)
        firstc = jnp.minimum(first, N - 1) + off
        o_ref[0] = jnp.broadcast_to(firstc, (Sblk, K))
        m_ref[...] = first
        for k in range(1, K):

            @pl.when(k < maxc)
            def _(k=k, o_ref=o_ref, v_ref=v_ref, m_ref=m_ref, firstc=firstc):
                v = v_ref[...]
                m = jnp.min(
                    jnp.where(v > m_ref[...], v, n32), axis=1, keepdims=True
                )
                col = jnp.where(m == n32, firstc, m + off)
                o_ref[0, :, k : k + 1] = col
                m_ref[...] = m


def _ballq(P, Q, r1, K1, r2, K2, Sblk):
    B, _, N = P.shape
    S = Q.shape[1]
    body = functools.partial(_ballq_body, N=N, r1=r1, r2=r2, K1=K1, K2=K2)
    g1, g2 = pl.pallas_call(
        body,
        grid=(B, S // Sblk),
        in_specs=[
            pl.BlockSpec((1, 8, N), lambda b, s: (b, 0, 0)),
            pl.BlockSpec((1, Sblk, 8), lambda b, s: (b, s, 0)),
        ],
        out_specs=(
            pl.BlockSpec((1, Sblk, K1), lambda b, s: (b, s, 0)),
            pl.BlockSpec((1, Sblk, K2), lambda b, s: (b, s, 0)),
        ),
        out_shape=(
            jax.ShapeDtypeStruct((B, S, K1), jnp.int32),
            jax.ShapeDtypeStruct((B, S, K2), jnp.int32),
        ),
        scratch_shapes=[
            pltpu.VMEM((Sblk, N), jnp.int32),
            pltpu.VMEM((Sblk, N), jnp.int32),
            pltpu.VMEM((Sblk, 1), jnp.int32),
            pltpu.VMEM((Sblk, 1), jnp.int32),
        ],
    )(P, Q)
    return g1, g2


def _atable0_body(pc_ref, w1_ref, b1_ref, w2_ref, b2_ref, a1_ref, a2_ref):
    P = pc_ref[0]
    Ft = jnp.concatenate([P, P[0:3]], axis=0)
    dn = (((0,), (0,)), ((), ()))
    a1_ref[...] = (
        lax.dot_general(Ft, w1_ref[...], dn, preferred_element_type=jnp.float32)
        + b1_ref[...]
    )
    a2_ref[...] = (
        lax.dot_general(Ft, w2_ref[...], dn, preferred_element_type=jnp.float32)
        + b2_ref[...]
    )


def _atable0(pc, W1, b1, W2, b2, blk):
    B, _, N = pc.shape
    C1 = W1.shape[1]
    C2 = W2.shape[1]
    nb = N // blk
    return pl.pallas_call(
        _atable0_body,
        grid=(B, nb),
        in_specs=[
            pl.BlockSpec((1, 9, blk), lambda b, n: (b, 0, n)),
            pl.BlockSpec((12, C1), lambda b, n: (0, 0)),
            pl.BlockSpec((1, C1), lambda b, n: (0, 0)),
            pl.BlockSpec((12, C2), lambda b, n: (0, 0)),
            pl.BlockSpec((1, C2), lambda b, n: (0, 0)),
        ],
        out_specs=(
            pl.BlockSpec((blk, C1), lambda b, n: (b * nb + n, 0)),
            pl.BlockSpec((blk, C2), lambda b, n: (b * nb + n, 0)),
        ),
        out_shape=(
            jax.ShapeDtypeStruct((B * N, C1), jnp.float32),
            jax.ShapeDtypeStruct((B * N, C2), jnp.float32),
        ),
    )(pc, W1, b1, W2, b2)


def _atable_body(p_ref, x_ref, wp1_ref, wx1_ref, b1_ref, wp2_ref, wx2_ref, b2_ref,
                 a1_ref, a2_ref):
    P = p_ref[...]
    X = x_ref[...]
    a1_ref[...] = (
        jnp.dot(P, wp1_ref[...], preferred_element_type=jnp.float32)
        + jnp.dot(X, wx1_ref[...], preferred_element_type=jnp.float32)
        + b1_ref[...]
    )
    a2_ref[...] = (
        jnp.dot(P, wp2_ref[...], preferred_element_type=jnp.float32)
        + jnp.dot(X, wx2_ref[...], preferred_element_type=jnp.float32)
        + b2_ref[...]
    )


def _atable(pts, xyzc, Wp1, Wx1, b1, Wp2, Wx2, b2, blk):
    R, Cin = pts.shape
    C1 = Wp1.shape[1]
    C2 = Wp2.shape[1]
    return pl.pallas_call(
        _atable_body,
        grid=(R // blk,),
        in_specs=[
            pl.BlockSpec((blk, Cin), lambda i: (i, 0)),
            pl.BlockSpec((blk, 8), lambda i: (i, 0)),
            pl.BlockSpec((Cin, C1), lambda i: (0, 0)),
            pl.BlockSpec((8, C1), lambda i: (0, 0)),
            pl.BlockSpec((1, C1), lambda i: (0, 0)),
            pl.BlockSpec((Cin, C2), lambda i: (0, 0)),
            pl.BlockSpec((8, C2), lambda i: (0, 0)),
            pl.BlockSpec((1, C2), lambda i: (0, 0)),
        ],
        out_specs=(
            pl.BlockSpec((blk, C1), lambda i: (i, 0)),
            pl.BlockSpec((blk, C2), lambda i: (i, 0)),
        ),
        out_shape=(
            jax.ShapeDtypeStruct((R, C1), jnp.float32),
            jax.ShapeDtypeStruct((R, C2), jnp.float32),
        ),
    )(pts, xyzc, Wp1, Wx1, b1, Wp2, Wx2, b2)


def _sc_gather(table, idx, R, C):
    info = plsc.get_sparse_core_info()
    NW = info.num_cores * info.num_subcores
    per_w = R // NW
    chunk = min(128, per_w)
    n_chunks = per_w // chunk
    nbuf = max(1, min(n_chunks, 4, 98000 // (chunk * C)))
    mesh = plsc.VectorSubcoreMesh(core_axis_name="c", subcore_axis_name="s")

    @functools.partial(
        pl.kernel,
        mesh=mesh,
        out_type=jax.ShapeDtypeStruct((R, C), jnp.float32),
        compiler_params=pltpu.CompilerParams(use_tc_tiling_on_sc=False),
        scratch_types=[
            pltpu.VMEM((n_chunks, chunk), jnp.int32),
            pltpu.VMEM((nbuf, chunk, C), jnp.float32),
            pltpu.SemaphoreType.DMA((nbuf,)),
            pltpu.SemaphoreType.DMA((nbuf,)),
        ],
    )
    def k(table_hbm, idx_hbm, out_hbm, idx_v, bufs, gsem, ssem):
        wid = lax.axis_index("s") * info.num_cores + lax.axis_index("c")
        base = wid * per_w
        pltpu.sync_copy(
            idx_hbm.at[pl.ds(wid * n_chunks, n_chunks), :], idx_v
        )
        gh = [None] * nbuf
        sh = [None] * nbuf

        def gather(j, b):
            return pltpu.async_copy(
                table_hbm.at[idx_v.at[j]], bufs.at[b], gsem.at[b]
            )

        def store(j, b):
            return pltpu.async_copy(
                bufs.at[b],
                out_hbm.at[pl.ds(base + j * chunk, chunk), :],
                ssem.at[b],
            )

        nstored = 0
        for j in range(n_chunks):
            b = j % nbuf
            if sh[b] is not None:
                sh[b].wait()
                sh[b] = None
            gh[b] = gather(j, b)
            if j >= nbuf - 1:
                bb = nstored % nbuf
                gh[bb].wait()
                sh[bb] = store(nstored, bb)
                nstored += 1
        while nstored < n_chunks:
            bb = nstored % nbuf
            gh[bb].wait()
            sh[bb] = store(nstored, bb)
            nstored += 1
        for b in range(nbuf):
            if sh[b] is not None:
                sh[b].wait()

    idx2d = idx.reshape(NW * n_chunks, chunk)
    return k(table, idx2d)


def _mlp_max_body(g_ref, q_ref, wq_ref, w2_ref, b2_ref, w3_ref, b3_ref, o_ref, *, K):
    G = g_ref[0]
    q = q_ref[0]
    Sblk = q.shape[0]
    C1 = G.shape[1]
    bq = jnp.dot(q, wq_ref[...], preferred_element_type=jnp.float32)
    H = jnp.maximum(G.reshape(Sblk, K, C1) - bq[:, None, :], 0.0)
    H = H.reshape(Sblk * K, C1)
    H = jnp.maximum(
        jnp.dot(H, w2_ref[...], preferred_element_type=jnp.float32) + b2_ref[...], 0.0
    )
    H = jnp.maximum(
        jnp.dot(H, w3_ref[...], preferred_element_type=jnp.float32) + b3_ref[...], 0.0
    )
    C3 = H.shape[1]
    o_ref[0] = jnp.max(H.reshape(Sblk, K, C3), axis=1)


def _mlp_max(Gf, Q, Wq, W2, b2, W3, b3, K, Sblk):
    B, SK, C1 = Gf.shape
    S = SK // K
    C2 = W2.shape[1]
    C3 = W3.shape[1]
    body = functools.partial(_mlp_max_body, K=K)
    return pl.pallas_call(
        body,
        grid=(B, S // Sblk),
        in_specs=[
            pl.BlockSpec((1, Sblk * K, C1), lambda b, s: (b, s, 0)),
            pl.BlockSpec((1, Sblk, 8), lambda b, s: (b, s, 0)),
            pl.BlockSpec((8, C1), lambda b, s: (0, 0)),
            pl.BlockSpec((C1, C2), lambda b, s: (0, 0)),
            pl.BlockSpec((1, C2), lambda b, s: (0, 0)),
            pl.BlockSpec((C2, C3), lambda b, s: (0, 0)),
            pl.BlockSpec((1, C3), lambda b, s: (0, 0)),
        ],
        out_specs=pl.BlockSpec((1, Sblk, C3), lambda b, s: (b, s, 0)),
        out_shape=jax.ShapeDtypeStruct((B, S, C3), jnp.float32),
    )(Gf, Q, Wq, W2, b2, W3, b3)


def _fp_body(x1_ref, x2_ref, p2_ref, *rest, n_layers, has_p1, has_cls, S):
    i = 0
    p1_ref = None
    if has_p1:
        p1_ref = rest[0]
        i = 1
    w_int = rest[i]
    w_p1 = rest[i + 1] if has_p1 else None
    b_1 = rest[i + 1 + (1 if has_p1 else 0)]
    rest = rest[i + 2 + (1 if has_p1 else 0) :]
    layer_ws = []
    for _ in range(n_layers - 1):
        layer_ws.append((rest[0], rest[1]))
        rest = rest[2:]
    cls_ws = None
    if has_cls:
        cls_ws = (rest[0], rest[1], rest[2], rest[3])
        rest = rest[4:]
    o_ref = rest[0]

    q = x1_ref[0]
    p = x2_ref[0]
    Nblk = q.shape[0]
    qn = jnp.sum(q * q, axis=1, keepdims=True)
    psq = p * p
    pn = lax.dot_general(
        jnp.ones((1, 8), jnp.float32),
        psq,
        (((1,), (1,)), ((), ())),
        preferred_element_type=jnp.float32,
    )
    cross = lax.dot_general(
        q, p, (((1,), (1,)), ((), ())), preferred_element_type=jnp.float32
    )
    d = qn + pn - 2.0 * cross
    iota = lax.broadcasted_iota(jnp.int32, (Nblk, S), 1)
    m1 = jnp.min(d, axis=1, keepdims=True)
    i1 = jnp.min(jnp.where(d == m1, iota, S), axis=1, keepdims=True)
    m2 = jnp.min(jnp.where(iota == i1, 1e30, d), axis=1, keepdims=True)
    i2 = jnp.min(
        jnp.where((d == m2) & (iota != i1), iota, S), axis=1, keepdims=True
    )
    ex = lambda: (iota == i1) | (iota == i2)
    m3 = jnp.min(jnp.where(ex(), 1e30, d), axis=1, keepdims=True)
    i3 = jnp.min(
        jnp.where((d == m3) & ~ex(), iota, S), axis=1, keepdims=True
    )
    r1_, r2_, r3_ = 1.0 / (m1 + 1e-8), 1.0 / (m2 + 1e-8), 1.0 / (m3 + 1e-8)
    tot = r1_ + r2_ + r3_
    Wm = jnp.where(
        iota == i1,
        r1_ / tot,
        jnp.where(iota == i2, r2_ / tot, jnp.where(iota == i3, r3_ / tot, 0.0)),
    )
    interp = jnp.dot(Wm, p2_ref[0], preferred_element_type=jnp.float32)
    acc = jnp.dot(interp, w_int[...], preferred_element_type=jnp.float32)
    if has_p1:
        acc = acc + jnp.dot(
            p1_ref[0], w_p1[...], preferred_element_type=jnp.float32
        )
    H = jnp.maximum(acc + b_1[...], 0.0)
    for w, b in layer_ws:
        H = jnp.maximum(
            jnp.dot(H, w[...], preferred_element_type=jnp.float32) + b[...], 0.0
        )
    if has_cls:
        wc1, bc1, wc2, bc2 = cls_ws
        H = jnp.maximum(
            jnp.dot(H, wc1[...], preferred_element_type=jnp.float32) + bc1[...], 0.0
        )
        logits = jnp.dot(H, wc2[...], preferred_element_type=jnp.float32) + bc2[...]
        mx = jnp.max(logits, axis=1, keepdims=True)
        lse = jnp.log(jnp.sum(jnp.exp(logits - mx), axis=1, keepdims=True)) + mx
        o_ref[0] = logits - lse
    else:
        o_ref[0] = H


def _fp(X1, X2, P2, P1, layers, cls, Nblk):
    B, N, _ = X1.shape
    S = X2.shape[1]
    C2 = P2.shape[2]
    has_p1 = P1 is not None
    C1 = P1.shape[2] if has_p1 else 0

    (W1, b1) = layers[0]
    w_int = W1[C1:, :]
    w_p1 = W1[:C1, :] if has_p1 else None

    args = [X1, X2, P2]
    specs = [
        pl.BlockSpec((1, Nblk, 8), lambda b, n: (b, n, 0)),
        pl.BlockSpec((1, S, 8), lambda b, n: (b, 0, 0)),
        pl.BlockSpec((1, S, C2), lambda b, n: (b, 0, 0)),
    ]
    if has_p1:
        args.append(P1)
        specs.append(pl.BlockSpec((1, Nblk, C1), lambda b, n: (b, n, 0)))
    args.append(w_int)
    specs.append(pl.BlockSpec(w_int.shape, lambda b, n: (0, 0)))
    if has_p1:
        args.append(w_p1)
        specs.append(pl.BlockSpec(w_p1.shape, lambda b, n: (0, 0)))
    args.append(b1.reshape(1, -1))
    specs.append(pl.BlockSpec((1, b1.shape[0]), lambda b, n: (0, 0)))
    for W, b in layers[1:]:
        args.append(W)
        specs.append(pl.BlockSpec(W.shape, lambda b, n: (0, 0)))
        args.append(b.reshape(1, -1))
        specs.append(pl.BlockSpec((1, b.shape[0]), lambda b, n: (0, 0)))
    if cls is not None:
        for arr in cls:
            args.append(arr)
            specs.append(pl.BlockSpec(arr.shape, lambda b, n: (0, 0)))
    Cout = 13 if cls is not None else layers[-1][0].shape[1]
    body = functools.partial(
        _fp_body,
        n_layers=len(layers),
        has_p1=has_p1,
        has_cls=cls is not None,
        S=S,
    )
    return pl.pallas_call(
        body,
        grid=(B, N // Nblk),
        in_specs=specs,
        out_specs=pl.BlockSpec((1, Nblk, Cout), lambda b, n: (b, n, 0)),
        out_shape=jax.ShapeDtypeStruct((B, N, Cout), jnp.float32),
    )(*args)


def _pad_lanes(x, mult):
    c = x.shape[-1]
    pad = (-c) % mult
    if pad == 0:
        return x
    return jnp.concatenate(
        [x, jnp.zeros(x.shape[:-1] + (pad,), x.dtype)], axis=-1
    )


def _coords_rows(X, Y, Z):
    B, N = X.shape
    return jnp.concatenate(
        [X[:, None], Y[:, None], Z[:, None], jnp.zeros((B, 5, N), X.dtype)], axis=1
    )


def _coords_cols(X, Y, Z):
    B, N = X.shape
    return jnp.concatenate(
        [X[..., None], Y[..., None], Z[..., None], jnp.zeros((B, N, 5), X.dtype)],
        axis=-1,
    )


def _sa_level(X, Y, Z, xyzc, pts, pc, cfg, branches, Sblk_bq, Sblk_mlp):
    S, (r1, r2), (K1, K2) = cfg
    B, N = X.shape
    Cin = 9 if pts is None else pts.shape[2]

    QX, QY, QZ = _fps(X, Y, Z, S)

    folded = [[_fold(p) for p in br] for br in branches]
    (W1a, b1a), (W2a, b2a), (W3a, b3a) = folded[0]
    (W1b, b1b), (W2b, b2b), (W3b, b3b) = folded[1]
    C1a = W1a.shape[0]
    C1b = W1b.shape[0]

    def wq(W1, C1p):
        return (
            jnp.zeros((8, C1p), jnp.float32)
            .at[:3, : W1.shape[0]]
            .set(W1[:, Cin : Cin + 3].T)
        )

    if pts is None:
        A1, A2 = _atable0(
            pc, W1a.T, b1a.reshape(1, -1), W1b.T, b1b.reshape(1, -1), blk=512
        )
    else:
        A1, A2 = _atable(
            pts.reshape(B * N, Cin),
            xyzc.reshape(B * N, 8),
            W1a[:, :Cin].T,
            wq(W1a, C1a),
            b1a.reshape(1, -1),
            W1b[:, :Cin].T,
            wq(W1b, C1b),
            b1b.reshape(1, -1),
            blk=min(512, B * N),
        )

    P = _coords_rows(X, Y, Z)
    Q = _coords_cols(QX, QY, QZ)
    g1, g2 = _ballq(P, Q, r1, K1, r2, K2, Sblk_bq)

    G1 = _sc_gather(A1, g1.reshape(-1), B * S * K1, C1a).reshape(B, S * K1, C1a)
    G2 = _sc_gather(A2, g2.reshape(-1), B * S * K2, C1b).reshape(B, S * K2, C1b)

    def wpad(W2, C1p):
        return jnp.zeros((C1p, W2.shape[0]), jnp.float32).at[: W2.shape[1]].set(
            W2.T
        )

    o1 = _mlp_max(
        G1, Q, wq(W1a, C1a), wpad(W2a, C1a), b2a.reshape(1, -1),
        W3a.T, b3a.reshape(1, -1), K1, min(Sblk_mlp, S),
    )
    o2 = _mlp_max(
        G2, Q, wq(W1b, C1b), wpad(W2b, C1b), b2b.reshape(1, -1),
        W3b.T, b3b.reshape(1, -1), K2, min(Sblk_mlp, S),
    )
    return QX, QY, QZ, Q, jnp.concatenate([o1, o2], axis=-1)


def kernel(point_cloud, params):
    pc = point_cloud
    B, C0, N0 = pc.shape
    X0 = pc[:, 0, :]
    Y0 = pc[:, 1, :]
    Z0 = pc[:, 2, :]
    x0c = _coords_cols(X0, Y0, Z0)

    QX1, QY1, QZ1, x1c, p1 = _sa_level(
        X0, Y0, Z0, x0c, None, pc, _SA_CFGS[0], params["sa1"],
        Sblk_bq=128, Sblk_mlp=64,
    )
    QX2, QY2, QZ2, x2c, p2 = _sa_level(
        QX1, QY1, QZ1, x1c, p1, None, _SA_CFGS[1], params["sa2"],
        Sblk_bq=128, Sblk_mlp=64,
    )
    QX3, QY3, QZ3, x3c, p3 = _sa_level(
        QX2, QY2, QZ2, x2c, p2, None, _SA_CFGS[2], params["sa3"],
        Sblk_bq=64, Sblk_mlp=64,
    )
    QX4, QY4, QZ4, x4c, p4 = _sa_level(
        QX3, QY3, QZ3, x3c, p3, None, _SA_CFGS[3], params["sa4"],
        Sblk_bq=16, Sblk_mlp=16,
    )

    def fold_layers(ps):
        out = []
        for p in ps:
            W, b = _fold(p)
            out.append((W.T, b))
        return out

    fp1l = fold_layers(params["fp1"])
    fp2l = fold_layers(params["fp2"])
    fp3l = fold_layers(params["fp3"])
    fp4l = fold_layers(params["fp4"])
    Wc1, bc1 = _fold(params["cls1"])
    cls = (
        Wc1.T,
        bc1.reshape(1, -1),
        params["cls2"]["W"].T,
        params["cls2"]["b"].reshape(1, -1),
    )

    u1 = _fp(x3c, x4c, p4, p3, fp1l, None, Nblk=64)
    u2 = _fp(x2c, x3c, u1, p2, fp2l, None, Nblk=128)
    u3 = _fp(x1c, x2c, u2, p1, fp3l, None, Nblk=256)
    pred = _fp(x0c, x1c, u3, None, fp4l, cls, Nblk=256)

    p4_out = jnp.transpose(p4, (0, 2, 1))
    return pred, p4_out


# speedup vs baseline: 27.7893x; 1.0036x over previous
import functools

import jax
import jax.numpy as jnp
import numpy as np
from jax import lax
from jax.experimental import pallas as pl
from jax.experimental.pallas import tpu as pltpu
from jax.experimental.pallas import tpu_sc as plsc

_SA_CFGS = [
    (1024, [0.05, 0.1], [16, 32]),
    (256, [0.1, 0.2], [16, 32]),
    (64, [0.2, 0.4], [16, 32]),
    (16, [0.4, 0.8], [16, 32]),
]
_B = 8
_N0 = 4096


def _fold(p):
    s = p["gamma"] / np.sqrt(1.0 + 1e-5)
    return p["W"] * s[:, None], p["b"] * s + p["beta"]


def _fps_body(x_ref, y_ref, z_ref, qx_ref, qy_ref, qz_ref):
    B, N = x_ref.shape
    S = qx_ref.shape[0]
    X = x_ref[...]
    Y = y_ref[...]
    Z = z_ref[...]
    iota = lax.broadcasted_iota(jnp.int32, (B, N), 1)
    eye = lax.broadcasted_iota(jnp.int32, (B, B), 0) == lax.broadcasted_iota(
        jnp.int32, (B, B), 1
    )

    def row(v):
        return jnp.sum(jnp.where(eye, v, 0.0), axis=0, keepdims=True)

    def body(i, state):
        far, dist = state
        onehot = iota == far
        cx = jnp.sum(jnp.where(onehot, X, 0.0), axis=1, keepdims=True)
        cy = jnp.sum(jnp.where(onehot, Y, 0.0), axis=1, keepdims=True)
        cz = jnp.sum(jnp.where(onehot, Z, 0.0), axis=1, keepdims=True)
        dx = X - cx
        dy = Y - cy
        dz = Z - cz
        d = dx * dx + dy * dy + dz * dz
        dist = jnp.minimum(dist, d)
        m = jnp.max(dist, axis=1, keepdims=True)
        far_new = jnp.min(
            jnp.where(dist == m, iota, N), axis=1, keepdims=True
        ).astype(jnp.int32)
        qx_ref[pl.ds(i, 1), :] = row(cx)
        qy_ref[pl.ds(i, 1), :] = row(cy)
        qz_ref[pl.ds(i, 1), :] = row(cz)
        return far_new, dist

    lax.fori_loop(
        0,
        S,
        body,
        (jnp.zeros((B, 1), jnp.int32), jnp.full((B, N), 1e10, jnp.float32)),
    )


def _fps(X, Y, Z, S):
    B, N = X.shape
    out = jax.ShapeDtypeStruct((S, B), jnp.float32)
    qx, qy, qz = pl.pallas_call(
        _fps_body,
        out_shape=(out, out, out),
    )(X, Y, Z)
    return qx.T, qy.T, qz.T


def _ballq_body(
    p_ref, q_ref, o1_ref, o2_ref, v1_ref, v2_ref, m1_ref, m2_ref, *, N, r1, r2, K1, K2
):
    P = p_ref[0]
    Q = q_ref[0]
    Sblk = Q.shape[0]
    pn = jnp.sum(P * P, axis=0, keepdims=True)
    qn = jnp.sum(Q * Q, axis=1, keepdims=True)
    cross = jnp.dot(Q, P, preferred_element_type=jnp.float32)
    d = qn + pn - 2.0 * cross
    iota32 = lax.broadcasted_iota(jnp.int32, (Sblk, N), 1)
    n32 = jnp.int32(N)
    v1_ref[...] = jnp.where(d <= r1 * r1, iota32, n32)
    v2_ref[...] = jnp.where(d <= r2 * r2, iota32, n32)
    off = pl.program_id(0) * N

    for K, v_ref, m_ref, o_ref in (
        (K1, v1_ref, m1_ref, o1_ref),
        (K2, v2_ref, m2_ref, o2_ref),
    ):
        v0 = v_ref[...]
        maxc = jnp.max(jnp.sum((v0 < n32).astype(jnp.int32), axis=1))
        first = jnp.min(v0, axis=1, keepdims=True)
        firstc = jnp.minimum(first, N - 1) + off
        o_ref[0] = jnp.broadcast_to(firstc, (Sblk, K))
        m_ref[...] = first
        for k in range(1, K):

            @pl.when(k < maxc)
            def _(k=k, o_ref=o_ref, v_ref=v_ref, m_ref=m_ref, firstc=firstc):
                v = v_ref[...]
                m = jnp.min(
                    jnp.where(v > m_ref[...], v, n32), axis=1, keepdims=True
                )
                col = jnp.where(m == n32, firstc, m + off)
                o_ref[0, :, k : k + 1] = col
                m_ref[...] = m


def _ballq(P, Q, r1, K1, r2, K2, Sblk):
    B, _, N = P.shape
    S = Q.shape[1]
    body = functools.partial(_ballq_body, N=N, r1=r1, r2=r2, K1=K1, K2=K2)
    g1, g2 = pl.pallas_call(
        body,
        grid=(B, S // Sblk),
        in_specs=[
            pl.BlockSpec((1, 8, N), lambda b, s: (b, 0, 0)),
            pl.BlockSpec((1, Sblk, 8), lambda b, s: (b, s, 0)),
        ],
        out_specs=(
            pl.BlockSpec((1, Sblk, K1), lambda b, s: (b, s, 0)),
            pl.BlockSpec((1, Sblk, K2), lambda b, s: (b, s, 0)),
        ),
        out_shape=(
            jax.ShapeDtypeStruct((B, S, K1), jnp.int32),
            jax.ShapeDtypeStruct((B, S, K2), jnp.int32),
        ),
        scratch_shapes=[
            pltpu.VMEM((Sblk, N), jnp.int32),
            pltpu.VMEM((Sblk, N), jnp.int32),
            pltpu.VMEM((Sblk, 1), jnp.int32),
            pltpu.VMEM((Sblk, 1), jnp.int32),
        ],
    )(P, Q)
    return g1, g2


def _atable0_body(pc_ref, w1_ref, b1_ref, w2_ref, b2_ref, a1_ref, a2_ref):
    P = pc_ref[0]
    Ft = jnp.concatenate([P, P[0:3]], axis=0)
    dn = (((0,), (0,)), ((), ()))
    a1_ref[...] = (
        lax.dot_general(Ft, w1_ref[...], dn, preferred_element_type=jnp.float32)
        + b1_ref[...]
    )
    a2_ref[...] = (
        lax.dot_general(Ft, w2_ref[...], dn, preferred_element_type=jnp.float32)
        + b2_ref[...]
    )


def _atable0(pc, W1, b1, W2, b2, blk):
    B, _, N = pc.shape
    C1 = W1.shape[1]
    C2 = W2.shape[1]
    nb = N // blk
    return pl.pallas_call(
        _atable0_body,
        grid=(B, nb),
        in_specs=[
            pl.BlockSpec((1, 9, blk), lambda b, n: (b, 0, n)),
            pl.BlockSpec((12, C1), lambda b, n: (0, 0)),
            pl.BlockSpec((1, C1), lambda b, n: (0, 0)),
            pl.BlockSpec((12, C2), lambda b, n: (0, 0)),
            pl.BlockSpec((1, C2), lambda b, n: (0, 0)),
        ],
        out_specs=(
            pl.BlockSpec((blk, C1), lambda b, n: (b * nb + n, 0)),
            pl.BlockSpec((blk, C2), lambda b, n: (b * nb + n, 0)),
        ),
        out_shape=(
            jax.ShapeDtypeStruct((B * N, C1), jnp.float32),
            jax.ShapeDtypeStruct((B * N, C2), jnp.float32),
        ),
    )(pc, W1, b1, W2, b2)


def _atable_body(p_ref, x_ref, wp1_ref, wx1_ref, b1_ref, wp2_ref, wx2_ref, b2_ref,
                 a1_ref, a2_ref):
    P = p_ref[...]
    X = x_ref[...]
    a1_ref[...] = (
        jnp.dot(P, wp1_ref[...], preferred_element_type=jnp.float32)
        + jnp.dot(X, wx1_ref[...], preferred_element_type=jnp.float32)
        + b1_ref[...]
    )
    a2_ref[...] = (
        jnp.dot(P, wp2_ref[...], preferred_element_type=jnp.float32)
        + jnp.dot(X, wx2_ref[...], preferred_element_type=jnp.float32)
        + b2_ref[...]
    )


def _atable(pts, xyzc, Wp1, Wx1, b1, Wp2, Wx2, b2, blk):
    R, Cin = pts.shape
    C1 = Wp1.shape[1]
    C2 = Wp2.shape[1]
    return pl.pallas_call(
        _atable_body,
        grid=(R // blk,),
        in_specs=[
            pl.BlockSpec((blk, Cin), lambda i: (i, 0)),
            pl.BlockSpec((blk, 8), lambda i: (i, 0)),
            pl.BlockSpec((Cin, C1), lambda i: (0, 0)),
            pl.BlockSpec((8, C1), lambda i: (0, 0)),
            pl.BlockSpec((1, C1), lambda i: (0, 0)),
            pl.BlockSpec((Cin, C2), lambda i: (0, 0)),
            pl.BlockSpec((8, C2), lambda i: (0, 0)),
            pl.BlockSpec((1, C2), lambda i: (0, 0)),
        ],
        out_specs=(
            pl.BlockSpec((blk, C1), lambda i: (i, 0)),
            pl.BlockSpec((blk, C2), lambda i: (i, 0)),
        ),
        out_shape=(
            jax.ShapeDtypeStruct((R, C1), jnp.float32),
            jax.ShapeDtypeStruct((R, C2), jnp.float32),
        ),
    )(pts, xyzc, Wp1, Wx1, b1, Wp2, Wx2, b2)


def _sc_gather(table, idx, R, C):
    info = plsc.get_sparse_core_info()
    NW = info.num_cores * info.num_subcores
    per_w = R // NW
    chunk = min(128, per_w)
    n_chunks = per_w // chunk
    G = max(1, min(n_chunks, 96000 // (chunk * C), 16))
    groups = []
    j = 0
    while j < n_chunks:
        groups.append((j, min(G, n_chunks - j)))
        j += G
    mesh = plsc.VectorSubcoreMesh(core_axis_name="c", subcore_axis_name="s")

    @functools.partial(
        pl.kernel,
        mesh=mesh,
        out_type=jax.ShapeDtypeStruct((R, C), jnp.float32),
        compiler_params=pltpu.CompilerParams(use_tc_tiling_on_sc=False),
        scratch_types=[
            pltpu.VMEM((n_chunks, chunk), jnp.int32),
            pltpu.VMEM((G * chunk, C), jnp.float32),
            pltpu.SemaphoreType.DMA,
        ],
    )
    def k(table_hbm, idx_hbm, out_hbm, idx_v, rows_v, sem):
        wid = lax.axis_index("s") * info.num_cores + lax.axis_index("c")
        base = wid * per_w
        pltpu.sync_copy(
            idx_hbm.at[pl.ds(wid * n_chunks, n_chunks), :], idx_v
        )
        for g0, glen in groups:
            copies = []
            for t in range(glen):
                copies.append(
                    pltpu.async_copy(
                        table_hbm.at[idx_v.at[g0 + t]],
                        rows_v.at[pl.ds(t * chunk, chunk), :],
                        sem,
                    )
                )
            for c in copies:
                c.wait()
            pltpu.sync_copy(
                rows_v.at[pl.ds(0, glen * chunk), :],
                out_hbm.at[pl.ds(base + g0 * chunk, glen * chunk), :],
            )

    idx2d = idx.reshape(NW * n_chunks, chunk)
    return k(table, idx2d)


def _mlp_max_body(g_ref, q_ref, wq_ref, w2_ref, b2_ref, w3_ref, b3_ref, o_ref, *, K):
    G = g_ref[0]
    q = q_ref[0]
    Sblk = q.shape[0]
    C1 = G.shape[1]
    bq = jnp.dot(q, wq_ref[...], preferred_element_type=jnp.float32)
    H = jnp.maximum(G.reshape(Sblk, K, C1) - bq[:, None, :], 0.0)
    H = H.reshape(Sblk * K, C1)
    H = jnp.maximum(
        jnp.dot(H, w2_ref[...], preferred_element_type=jnp.float32) + b2_ref[...], 0.0
    )
    H = jnp.maximum(
        jnp.dot(H, w3_ref[...], preferred_element_type=jnp.float32) + b3_ref[...], 0.0
    )
    C3 = H.shape[1]
    o_ref[0] = jnp.max(H.reshape(Sblk, K, C3), axis=1)


def _mlp_max(Gf, Q, Wq, W2, b2, W3, b3, K, Sblk):
    B, SK, C1 = Gf.shape
    S = SK // K
    C2 = W2.shape[1]
    C3 = W3.shape[1]
    body = functools.partial(_mlp_max_body, K=K)
    return pl.pallas_call(
        body,
        grid=(B, S // Sblk),
        in_specs=[
            pl.BlockSpec((1, Sblk * K, C1), lambda b, s: (b, s, 0)),
            pl.BlockSpec((1, Sblk, 8), lambda b, s: (b, s, 0)),
            pl.BlockSpec((8, C1), lambda b, s: (0, 0)),
            pl.BlockSpec((C1, C2), lambda b, s: (0, 0)),
            pl.BlockSpec((1, C2), lambda b, s: (0, 0)),
            pl.BlockSpec((C2, C3), lambda b, s: (0, 0)),
            pl.BlockSpec((1, C3), lambda b, s: (0, 0)),
        ],
        out_specs=pl.BlockSpec((1, Sblk, C3), lambda b, s: (b, s, 0)),
        out_shape=jax.ShapeDtypeStruct((B, S, C3), jnp.float32),
    )(Gf, Q, Wq, W2, b2, W3, b3)


def _fp_body(x1_ref, x2_ref, p2_ref, *rest, n_layers, has_p1, has_cls, S):
    i = 0
    p1_ref = None
    if has_p1:
        p1_ref = rest[0]
        i = 1
    w_int = rest[i]
    w_p1 = rest[i + 1] if has_p1 else None
    b_1 = rest[i + 1 + (1 if has_p1 else 0)]
    rest = rest[i + 2 + (1 if has_p1 else 0) :]
    layer_ws = []
    for _ in range(n_layers - 1):
        layer_ws.append((rest[0], rest[1]))
        rest = rest[2:]
    cls_ws = None
    if has_cls:
        cls_ws = (rest[0], rest[1], rest[2], rest[3])
        rest = rest[4:]
    o_ref = rest[0]

    q = x1_ref[0]
    p = x2_ref[0]
    Nblk = q.shape[0]
    qn = jnp.sum(q * q, axis=1, keepdims=True)
    psq = p * p
    pn = lax.dot_general(
        jnp.ones((1, 8), jnp.float32),
        psq,
        (((1,), (1,)), ((), ())),
        preferred_element_type=jnp.float32,
    )
    cross = lax.dot_general(
        q, p, (((1,), (1,)), ((), ())), preferred_element_type=jnp.float32
    )
    d = qn + pn - 2.0 * cross
    iota = lax.broadcasted_iota(jnp.int32, (Nblk, S), 1)
    m1 = jnp.min(d, axis=1, keepdims=True)
    i1 = jnp.min(jnp.where(d == m1, iota, S), axis=1, keepdims=True)
    m2 = jnp.min(jnp.where(iota == i1, 1e30, d), axis=1, keepdims=True)
    i2 = jnp.min(
        jnp.where((d == m2) & (iota != i1), iota, S), axis=1, keepdims=True
    )
    ex = lambda: (iota == i1) | (iota == i2)
    m3 = jnp.min(jnp.where(ex(), 1e30, d), axis=1, keepdims=True)
    i3 = jnp.min(
        jnp.where((d == m3) & ~ex(), iota, S), axis=1, keepdims=True
    )
    r1_, r2_, r3_ = 1.0 / (m1 + 1e-8), 1.0 / (m2 + 1e-8), 1.0 / (m3 + 1e-8)
    tot = r1_ + r2_ + r3_
    Wm = jnp.where(
        iota == i1,
        r1_ / tot,
        jnp.where(iota == i2, r2_ / tot, jnp.where(iota == i3, r3_ / tot, 0.0)),
    )
    interp = jnp.dot(Wm, p2_ref[0], preferred_element_type=jnp.float32)
    acc = jnp.dot(interp, w_int[...], preferred_element_type=jnp.float32)
    if has_p1:
        acc = acc + jnp.dot(
            p1_ref[0], w_p1[...], preferred_element_type=jnp.float32
        )
    H = jnp.maximum(acc + b_1[...], 0.0)
    for w, b in layer_ws:
        H = jnp.maximum(
            jnp.dot(H, w[...], preferred_element_type=jnp.float32) + b[...], 0.0
        )
    if has_cls:
        wc1, bc1, wc2, bc2 = cls_ws
        H = jnp.maximum(
            jnp.dot(H, wc1[...], preferred_element_type=jnp.float32) + bc1[...], 0.0
        )
        logits = jnp.dot(H, wc2[...], preferred_element_type=jnp.float32) + bc2[...]
        mx = jnp.max(logits, axis=1, keepdims=True)
        lse = jnp.log(jnp.sum(jnp.exp(logits - mx), axis=1, keepdims=True)) + mx
        o_ref[0] = logits - lse
    else:
        o_ref[0] = H


def _fp(X1, X2, P2, P1, layers, cls, Nblk):
    B, N, _ = X1.shape
    S = X2.shape[1]
    C2 = P2.shape[2]
    has_p1 = P1 is not None
    C1 = P1.shape[2] if has_p1 else 0

    (W1, b1) = layers[0]
    w_int = W1[C1:, :]
    w_p1 = W1[:C1, :] if has_p1 else None

    args = [X1, X2, P2]
    specs = [
        pl.BlockSpec((1, Nblk, 8), lambda b, n: (b, n, 0)),
        pl.BlockSpec((1, S, 8), lambda b, n: (b, 0, 0)),
        pl.BlockSpec((1, S, C2), lambda b, n: (b, 0, 0)),
    ]
    if has_p1:
        args.append(P1)
        specs.append(pl.BlockSpec((1, Nblk, C1), lambda b, n: (b, n, 0)))
    args.append(w_int)
    specs.append(pl.BlockSpec(w_int.shape, lambda b, n: (0, 0)))
    if has_p1:
        args.append(w_p1)
        specs.append(pl.BlockSpec(w_p1.shape, lambda b, n: (0, 0)))
    args.append(b1.reshape(1, -1))
    specs.append(pl.BlockSpec((1, b1.shape[0]), lambda b, n: (0, 0)))
    for W, b in layers[1:]:
        args.append(W)
        specs.append(pl.BlockSpec(W.shape, lambda b, n: (0, 0)))
        args.append(b.reshape(1, -1))
        specs.append(pl.BlockSpec((1, b.shape[0]), lambda b, n: (0, 0)))
    if cls is not None:
        for arr in cls:
            args.append(arr)
            specs.append(pl.BlockSpec(arr.shape, lambda b, n: (0, 0)))
    Cout = 13 if cls is not None else layers[-1][0].shape[1]
    body = functools.partial(
        _fp_body,
        n_layers=len(layers),
        has_p1=has_p1,
        has_cls=cls is not None,
        S=S,
    )
    return pl.pallas_call(
        body,
        grid=(B, N // Nblk),
        in_specs=specs,
        out_specs=pl.BlockSpec((1, Nblk, Cout), lambda b, n: (b, n, 0)),
        out_shape=jax.ShapeDtypeStruct((B, N, Cout), jnp.float32),
    )(*args)


def _pad_lanes(x, mult):
    c = x.shape[-1]
    pad = (-c) % mult
    if pad == 0:
        return x
    return jnp.concatenate(
        [x, jnp.zeros(x.shape[:-1] + (pad,), x.dtype)], axis=-1
    )


def _coords_rows(X, Y, Z):
    B, N = X.shape
    return jnp.concatenate(
        [X[:, None], Y[:, None], Z[:, None], jnp.zeros((B, 5, N), X.dtype)], axis=1
    )


def _coords_cols(X, Y, Z):
    B, N = X.shape
    return jnp.concatenate(
        [X[..., None], Y[..., None], Z[..., None], jnp.zeros((B, N, 5), X.dtype)],
        axis=-1,
    )


def _sa_level(X, Y, Z, xyzc, pts, pc, cfg, branches, Sblk_bq, Sblk_mlp):
    S, (r1, r2), (K1, K2) = cfg
    B, N = X.shape
    Cin = 9 if pts is None else pts.shape[2]

    QX, QY, QZ = _fps(X, Y, Z, S)

    folded = [[_fold(p) for p in br] for br in branches]
    (W1a, b1a), (W2a, b2a), (W3a, b3a) = folded[0]
    (W1b, b1b), (W2b, b2b), (W3b, b3b) = folded[1]
    C1a = W1a.shape[0]
    C1b = W1b.shape[0]

    def wq(W1, C1p):
        return (
            jnp.zeros((8, C1p), jnp.float32)
            .at[:3, : W1.shape[0]]
            .set(W1[:, Cin : Cin + 3].T)
        )

    if pts is None:
        A1, A2 = _atable0(
            pc, W1a.T, b1a.reshape(1, -1), W1b.T, b1b.reshape(1, -1), blk=512
        )
    else:
        A1, A2 = _atable(
            pts.reshape(B * N, Cin),
            xyzc.reshape(B * N, 8),
            W1a[:, :Cin].T,
            wq(W1a, C1a),
            b1a.reshape(1, -1),
            W1b[:, :Cin].T,
            wq(W1b, C1b),
            b1b.reshape(1, -1),
            blk=min(512, B * N),
        )

    P = _coords_rows(X, Y, Z)
    Q = _coords_cols(QX, QY, QZ)
    g1, g2 = _ballq(P, Q, r1, K1, r2, K2, Sblk_bq)

    G1 = _sc_gather(A1, g1.reshape(-1), B * S * K1, C1a).reshape(B, S * K1, C1a)
    G2 = _sc_gather(A2, g2.reshape(-1), B * S * K2, C1b).reshape(B, S * K2, C1b)

    def wpad(W2, C1p):
        return jnp.zeros((C1p, W2.shape[0]), jnp.float32).at[: W2.shape[1]].set(
            W2.T
        )

    o1 = _mlp_max(
        G1, Q, wq(W1a, C1a), wpad(W2a, C1a), b2a.reshape(1, -1),
        W3a.T, b3a.reshape(1, -1), K1, min(Sblk_mlp, S),
    )
    o2 = _mlp_max(
        G2, Q, wq(W1b, C1b), wpad(W2b, C1b), b2b.reshape(1, -1),
        W3b.T, b3b.reshape(1, -1), K2, min(Sblk_mlp, S),
    )
    return QX, QY, QZ, Q, jnp.concatenate([o1, o2], axis=-1)


def kernel(point_cloud, params):
    pc = point_cloud
    B, C0, N0 = pc.shape
    X0 = pc[:, 0, :]
    Y0 = pc[:, 1, :]
    Z0 = pc[:, 2, :]
    x0c = _coords_cols(X0, Y0, Z0)

    QX1, QY1, QZ1, x1c, p1 = _sa_level(
        X0, Y0, Z0, x0c, None, pc, _SA_CFGS[0], params["sa1"],
        Sblk_bq=128, Sblk_mlp=64,
    )
    QX2, QY2, QZ2, x2c, p2 = _sa_level(
        QX1, QY1, QZ1, x1c, p1, None, _SA_CFGS[1], params["sa2"],
        Sblk_bq=128, Sblk_mlp=64,
    )
    QX3, QY3, QZ3, x3c, p3 = _sa_level(
        QX2, QY2, QZ2, x2c, p2, None, _SA_CFGS[2], params["sa3"],
        Sblk_bq=64, Sblk_mlp=64,
    )
    QX4, QY4, QZ4, x4c, p4 = _sa_level(
        QX3, QY3, QZ3, x3c, p3, None, _SA_CFGS[3], params["sa4"],
        Sblk_bq=16, Sblk_mlp=16,
    )

    def fold_layers(ps):
        out = []
        for p in ps:
            W, b = _fold(p)
            out.append((W.T, b))
        return out

    fp1l = fold_layers(params["fp1"])
    fp2l = fold_layers(params["fp2"])
    fp3l = fold_layers(params["fp3"])
    fp4l = fold_layers(params["fp4"])
    Wc1, bc1 = _fold(params["cls1"])
    cls = (
        Wc1.T,
        bc1.reshape(1, -1),
        params["cls2"]["W"].T,
        params["cls2"]["b"].reshape(1, -1),
    )

    u1 = _fp(x3c, x4c, p4, p3, fp1l, None, Nblk=64)
    u2 = _fp(x2c, x3c, u1, p2, fp2l, None, Nblk=128)
    u3 = _fp(x1c, x2c, u2, p1, fp3l, None, Nblk=256)
    pred = _fp(x0c, x1c, u3, None, fp4l, cls, Nblk=256)

    p4_out = jnp.transpose(p4, (0, 2, 1))
    return pred, p4_out


# speedup vs baseline: 28.1754x; 1.0139x over previous
import functools

import jax
import jax.numpy as jnp
import numpy as np
from jax import lax
from jax.experimental import pallas as pl
from jax.experimental.pallas import tpu as pltpu
from jax.experimental.pallas import tpu_sc as plsc

_SA_CFGS = [
    (1024, [0.05, 0.1], [16, 32]),
    (256, [0.1, 0.2], [16, 32]),
    (64, [0.2, 0.4], [16, 32]),
    (16, [0.4, 0.8], [16, 32]),
]
_B = 8
_N0 = 4096


def _fold(p):
    s = p["gamma"] / np.sqrt(1.0 + 1e-5)
    return p["W"] * s[:, None], p["b"] * s + p["beta"]


def _fps_body(x_ref, y_ref, z_ref, qx_ref, qy_ref, qz_ref):
    B, N = x_ref.shape
    S = qx_ref.shape[0]
    X = x_ref[...]
    Y = y_ref[...]
    Z = z_ref[...]
    iota = lax.broadcasted_iota(jnp.int32, (B, N), 1)
    eye = lax.broadcasted_iota(jnp.int32, (B, B), 0) == lax.broadcasted_iota(
        jnp.int32, (B, B), 1
    )

    def row(v):
        return jnp.sum(jnp.where(eye, v, 0.0), axis=0, keepdims=True)

    def body(i, state):
        far, dist = state
        onehot = iota == far
        cx = jnp.sum(jnp.where(onehot, X, 0.0), axis=1, keepdims=True)
        cy = jnp.sum(jnp.where(onehot, Y, 0.0), axis=1, keepdims=True)
        cz = jnp.sum(jnp.where(onehot, Z, 0.0), axis=1, keepdims=True)
        dx = X - cx
        dy = Y - cy
        dz = Z - cz
        d = dx * dx + dy * dy + dz * dz
        dist = jnp.minimum(dist, d)
        m = jnp.max(dist, axis=1, keepdims=True)
        far_new = jnp.min(
            jnp.where(dist == m, iota, N), axis=1, keepdims=True
        ).astype(jnp.int32)
        qx_ref[pl.ds(i, 1), :] = row(cx)
        qy_ref[pl.ds(i, 1), :] = row(cy)
        qz_ref[pl.ds(i, 1), :] = row(cz)
        return far_new, dist

    lax.fori_loop(
        0,
        S,
        body,
        (jnp.zeros((B, 1), jnp.int32), jnp.full((B, N), 1e10, jnp.float32)),
    )


def _fps(X, Y, Z, S):
    B, N = X.shape
    out = jax.ShapeDtypeStruct((S, B), jnp.float32)
    qx, qy, qz = pl.pallas_call(
        _fps_body,
        out_shape=(out, out, out),
    )(X, Y, Z)
    return qx.T, qy.T, qz.T


def _ballq_body(
    p_ref, q_ref, o1_ref, o2_ref, v1_ref, v2_ref, m1_ref, m2_ref, *, N, r1, r2, K1, K2
):
    P = p_ref[0]
    Q = q_ref[0]
    Sblk = Q.shape[0]
    pn = jnp.sum(P * P, axis=0, keepdims=True)
    qn = jnp.sum(Q * Q, axis=1, keepdims=True)
    cross = jnp.dot(Q, P, preferred_element_type=jnp.float32)
    d = qn + pn - 2.0 * cross
    iota32 = lax.broadcasted_iota(jnp.int32, (Sblk, N), 1)
    n32 = jnp.int32(N)
    v1_ref[...] = jnp.where(d <= r1 * r1, iota32, n32)
    v2_ref[...] = jnp.where(d <= r2 * r2, iota32, n32)
    off = pl.program_id(0) * N

    for K, v_ref, m_ref, o_ref in (
        (K1, v1_ref, m1_ref, o1_ref),
        (K2, v2_ref, m2_ref, o2_ref),
    ):
        v0 = v_ref[...]
        maxc = jnp.max(jnp.sum((v0 < n32).astype(jnp.int32), axis=1))
        first = jnp.min(v0, axis=1, keepdims=True)
        firstc = jnp.minimum(first, N - 1) + off
        o_ref[0] = jnp.broadcast_to(firstc, (Sblk, K))
        m_ref[...] = first
        for k in range(1, K):

            @pl.when(k < maxc)
            def _(k=k, o_ref=o_ref, v_ref=v_ref, m_ref=m_ref, firstc=firstc):
                v = v_ref[...]
                m = jnp.min(
                    jnp.where(v > m_ref[...], v, n32), axis=1, keepdims=True
                )
                col = jnp.where(m == n32, firstc, m + off)
                o_ref[0, :, k : k + 1] = col
                m_ref[...] = m


def _ballq(P, Q, r1, K1, r2, K2, Sblk):
    B, _, N = P.shape
    S = Q.shape[1]
    body = functools.partial(_ballq_body, N=N, r1=r1, r2=r2, K1=K1, K2=K2)
    g1, g2 = pl.pallas_call(
        body,
        grid=(B, S // Sblk),
        in_specs=[
            pl.BlockSpec((1, 8, N), lambda b, s: (b, 0, 0)),
            pl.BlockSpec((1, Sblk, 8), lambda b, s: (b, s, 0)),
        ],
        out_specs=(
            pl.BlockSpec((1, Sblk, K1), lambda b, s: (b, s, 0)),
            pl.BlockSpec((1, Sblk, K2), lambda b, s: (b, s, 0)),
        ),
        out_shape=(
            jax.ShapeDtypeStruct((B, S, K1), jnp.int32),
            jax.ShapeDtypeStruct((B, S, K2), jnp.int32),
        ),
        scratch_shapes=[
            pltpu.VMEM((Sblk, N), jnp.int32),
            pltpu.VMEM((Sblk, N), jnp.int32),
            pltpu.VMEM((Sblk, 1), jnp.int32),
            pltpu.VMEM((Sblk, 1), jnp.int32),
        ],
    )(P, Q)
    return g1, g2


def _atable0_body(pc_ref, w1_ref, b1_ref, w2_ref, b2_ref, a1_ref, a2_ref):
    P = pc_ref[0]
    Ft = jnp.concatenate([P, P[0:3]], axis=0)
    dn = (((0,), (0,)), ((), ()))
    a1_ref[...] = (
        lax.dot_general(Ft, w1_ref[...], dn, preferred_element_type=jnp.float32)
        + b1_ref[...]
    )
    a2_ref[...] = (
        lax.dot_general(Ft, w2_ref[...], dn, preferred_element_type=jnp.float32)
        + b2_ref[...]
    )


def _atable0(pc, W1, b1, W2, b2, blk):
    B, _, N = pc.shape
    C1 = W1.shape[1]
    C2 = W2.shape[1]
    nb = N // blk
    return pl.pallas_call(
        _atable0_body,
        grid=(B, nb),
        in_specs=[
            pl.BlockSpec((1, 9, blk), lambda b, n: (b, 0, n)),
            pl.BlockSpec((12, C1), lambda b, n: (0, 0)),
            pl.BlockSpec((1, C1), lambda b, n: (0, 0)),
            pl.BlockSpec((12, C2), lambda b, n: (0, 0)),
            pl.BlockSpec((1, C2), lambda b, n: (0, 0)),
        ],
        out_specs=(
            pl.BlockSpec((blk, C1), lambda b, n: (b * nb + n, 0)),
            pl.BlockSpec((blk, C2), lambda b, n: (b * nb + n, 0)),
        ),
        out_shape=(
            jax.ShapeDtypeStruct((B * N, C1), jnp.float32),
            jax.ShapeDtypeStruct((B * N, C2), jnp.float32),
        ),
    )(pc, W1, b1, W2, b2)


def _atable_body(p_ref, x_ref, wp1_ref, wx1_ref, b1_ref, wp2_ref, wx2_ref, b2_ref,
                 a1_ref, a2_ref):
    P = p_ref[...]
    X = x_ref[...]
    a1_ref[...] = (
        jnp.dot(P, wp1_ref[...], preferred_element_type=jnp.float32)
        + jnp.dot(X, wx1_ref[...], preferred_element_type=jnp.float32)
        + b1_ref[...]
    )
    a2_ref[...] = (
        jnp.dot(P, wp2_ref[...], preferred_element_type=jnp.float32)
        + jnp.dot(X, wx2_ref[...], preferred_element_type=jnp.float32)
        + b2_ref[...]
    )


def _atable(pts, xyzc, Wp1, Wx1, b1, Wp2, Wx2, b2, blk):
    R, Cin = pts.shape
    C1 = Wp1.shape[1]
    C2 = Wp2.shape[1]
    return pl.pallas_call(
        _atable_body,
        grid=(R // blk,),
        in_specs=[
            pl.BlockSpec((blk, Cin), lambda i: (i, 0)),
            pl.BlockSpec((blk, 8), lambda i: (i, 0)),
            pl.BlockSpec((Cin, C1), lambda i: (0, 0)),
            pl.BlockSpec((8, C1), lambda i: (0, 0)),
            pl.BlockSpec((1, C1), lambda i: (0, 0)),
            pl.BlockSpec((Cin, C2), lambda i: (0, 0)),
            pl.BlockSpec((8, C2), lambda i: (0, 0)),
            pl.BlockSpec((1, C2), lambda i: (0, 0)),
        ],
        out_specs=(
            pl.BlockSpec((blk, C1), lambda i: (i, 0)),
            pl.BlockSpec((blk, C2), lambda i: (i, 0)),
        ),
        out_shape=(
            jax.ShapeDtypeStruct((R, C1), jnp.float32),
            jax.ShapeDtypeStruct((R, C2), jnp.float32),
        ),
    )(pts, xyzc, Wp1, Wx1, b1, Wp2, Wx2, b2)


def _sc_gather(table, idx, R, C):
    info = plsc.get_sparse_core_info()
    NW = info.num_cores * info.num_subcores
    per_w = R // NW
    chunk = min(128, per_w)
    n_chunks = per_w // chunk
    G = max(1, min(n_chunks, 96000 // (chunk * C), 16))
    groups = []
    j = 0
    while j < n_chunks:
        groups.append((j, min(G, n_chunks - j)))
        j += G
    mesh = plsc.VectorSubcoreMesh(core_axis_name="c", subcore_axis_name="s")

    @functools.partial(
        pl.kernel,
        mesh=mesh,
        out_type=jax.ShapeDtypeStruct((R, C), jnp.float32),
        compiler_params=pltpu.CompilerParams(use_tc_tiling_on_sc=False),
        scratch_types=[
            pltpu.VMEM((n_chunks, chunk), jnp.int32),
            pltpu.VMEM((G * chunk, C), jnp.float32),
            pltpu.SemaphoreType.DMA,
        ],
    )
    def k(table_hbm, idx_hbm, out_hbm, idx_v, rows_v, sem):
        wid = lax.axis_index("s") * info.num_cores + lax.axis_index("c")
        base = wid * per_w
        pltpu.sync_copy(
            idx_hbm.at[pl.ds(wid * n_chunks, n_chunks), :], idx_v
        )
        for g0, glen in groups:
            copies = []
            for t in range(glen):
                copies.append(
                    pltpu.async_copy(
                        table_hbm.at[idx_v.at[g0 + t]],
                        rows_v.at[pl.ds(t * chunk, chunk), :],
                        sem,
                    )
                )
            for c in copies:
                c.wait()
            pltpu.sync_copy(
                rows_v.at[pl.ds(0, glen * chunk), :],
                out_hbm.at[pl.ds(base + g0 * chunk, glen * chunk), :],
            )

    idx2d = idx.reshape(NW * n_chunks, chunk)
    return k(table, idx2d)


def _mlp_max_body(g_ref, q_ref, wq_ref, w2_ref, b2_ref, w3_ref, b3_ref, o_ref, *, K):
    G = g_ref[0]
    q = q_ref[0]
    Sblk = q.shape[0]
    C1 = G.shape[1]
    bq = jnp.dot(q, wq_ref[...], preferred_element_type=jnp.float32)
    H = jnp.maximum(G.reshape(Sblk, K, C1) - bq[:, None, :], 0.0)
    H = H.reshape(Sblk * K, C1)
    H = jnp.maximum(
        jnp.dot(H, w2_ref[...], preferred_element_type=jnp.float32) + b2_ref[...], 0.0
    )
    H = jnp.maximum(
        jnp.dot(H, w3_ref[...], preferred_element_type=jnp.float32) + b3_ref[...], 0.0
    )
    C3 = H.shape[1]
    o_ref[0] = jnp.max(H.reshape(Sblk, K, C3), axis=1)


def _mlp_max(Gf, Q, Wq, W2, b2, W3, b3, K, Sblk):
    B, SK, C1 = Gf.shape
    S = SK // K
    C2 = W2.shape[1]
    C3 = W3.shape[1]
    body = functools.partial(_mlp_max_body, K=K)
    return pl.pallas_call(
        body,
        grid=(B, S // Sblk),
        in_specs=[
            pl.BlockSpec((1, Sblk * K, C1), lambda b, s: (b, s, 0)),
            pl.BlockSpec((1, Sblk, 8), lambda b, s: (b, s, 0)),
            pl.BlockSpec((8, C1), lambda b, s: (0, 0)),
            pl.BlockSpec((C1, C2), lambda b, s: (0, 0)),
            pl.BlockSpec((1, C2), lambda b, s: (0, 0)),
            pl.BlockSpec((C2, C3), lambda b, s: (0, 0)),
            pl.BlockSpec((1, C3), lambda b, s: (0, 0)),
        ],
        out_specs=pl.BlockSpec((1, Sblk, C3), lambda b, s: (b, s, 0)),
        out_shape=jax.ShapeDtypeStruct((B, S, C3), jnp.float32),
    )(Gf, Q, Wq, W2, b2, W3, b3)


def _fp_body(x1_ref, x2_ref, p2_ref, *rest, n_layers, has_p1, has_cls, S):
    i = 0
    p1_ref = None
    if has_p1:
        p1_ref = rest[0]
        i = 1
    w_int = rest[i]
    w_p1 = rest[i + 1] if has_p1 else None
    b_1 = rest[i + 1 + (1 if has_p1 else 0)]
    rest = rest[i + 2 + (1 if has_p1 else 0) :]
    layer_ws = []
    for _ in range(n_layers - 1):
        layer_ws.append((rest[0], rest[1]))
        rest = rest[2:]
    cls_ws = None
    if has_cls:
        cls_ws = (rest[0], rest[1], rest[2], rest[3])
        rest = rest[4:]
    o_ref = rest[0]

    q = x1_ref[0]
    p = x2_ref[0]
    Nblk = q.shape[0]
    qn = jnp.sum(q * q, axis=1, keepdims=True)
    psq = p * p
    pn = lax.dot_general(
        jnp.ones((1, 8), jnp.float32),
        psq,
        (((1,), (1,)), ((), ())),
        preferred_element_type=jnp.float32,
    )
    cross = lax.dot_general(
        q, p, (((1,), (1,)), ((), ())), preferred_element_type=jnp.float32
    )
    d = qn + pn - 2.0 * cross
    iota = lax.broadcasted_iota(jnp.int32, (Nblk, S), 1)
    vals = d
    rs = []
    ohs = []
    for _ in range(3):
        m = jnp.min(vals, axis=1, keepdims=True)
        isel = jnp.min(jnp.where(vals == m, iota, S), axis=1, keepdims=True)
        oh = iota == isel
        rs.append(1.0 / (m + 1e-8))
        ohs.append(oh)
        vals = jnp.where(oh, 1e30, vals)
    tot = rs[0] + rs[1] + rs[2]
    Wm = (
        jnp.where(ohs[0], rs[0] / tot, 0.0)
        + jnp.where(ohs[1], rs[1] / tot, 0.0)
        + jnp.where(ohs[2], rs[2] / tot, 0.0)
    )
    interp = jnp.dot(Wm, p2_ref[0], preferred_element_type=jnp.float32)
    acc = jnp.dot(interp, w_int[...], preferred_element_type=jnp.float32)
    if has_p1:
        acc = acc + jnp.dot(
            p1_ref[0], w_p1[...], preferred_element_type=jnp.float32
        )
    H = jnp.maximum(acc + b_1[...], 0.0)
    for w, b in layer_ws:
        H = jnp.maximum(
            jnp.dot(H, w[...], preferred_element_type=jnp.float32) + b[...], 0.0
        )
    if has_cls:
        wc1, bc1, wc2, bc2 = cls_ws
        H = jnp.maximum(
            jnp.dot(H, wc1[...], preferred_element_type=jnp.float32) + bc1[...], 0.0
        )
        logits = jnp.dot(H, wc2[...], preferred_element_type=jnp.float32) + bc2[...]
        mx = jnp.max(logits, axis=1, keepdims=True)
        lse = jnp.log(jnp.sum(jnp.exp(logits - mx), axis=1, keepdims=True)) + mx
        o_ref[0] = logits - lse
    else:
        o_ref[0] = H


def _fp(X1, X2, P2, P1, layers, cls, Nblk):
    B, N, _ = X1.shape
    S = X2.shape[1]
    C2 = P2.shape[2]
    has_p1 = P1 is not None
    C1 = P1.shape[2] if has_p1 else 0

    (W1, b1) = layers[0]
    w_int = W1[C1:, :]
    w_p1 = W1[:C1, :] if has_p1 else None

    args = [X1, X2, P2]
    specs = [
        pl.BlockSpec((1, Nblk, 8), lambda b, n: (b, n, 0)),
        pl.BlockSpec((1, S, 8), lambda b, n: (b, 0, 0)),
        pl.BlockSpec((1, S, C2), lambda b, n: (b, 0, 0)),
    ]
    if has_p1:
        args.append(P1)
        specs.append(pl.BlockSpec((1, Nblk, C1), lambda b, n: (b, n, 0)))
    args.append(w_int)
    specs.append(pl.BlockSpec(w_int.shape, lambda b, n: (0, 0)))
    if has_p1:
        args.append(w_p1)
        specs.append(pl.BlockSpec(w_p1.shape, lambda b, n: (0, 0)))
    args.append(b1.reshape(1, -1))
    specs.append(pl.BlockSpec((1, b1.shape[0]), lambda b, n: (0, 0)))
    for W, b in layers[1:]:
        args.append(W)
        specs.append(pl.BlockSpec(W.shape, lambda b, n: (0, 0)))
        args.append(b.reshape(1, -1))
        specs.append(pl.BlockSpec((1, b.shape[0]), lambda b, n: (0, 0)))
    if cls is not None:
        for arr in cls:
            args.append(arr)
            specs.append(pl.BlockSpec(arr.shape, lambda b, n: (0, 0)))
    Cout = 13 if cls is not None else layers[-1][0].shape[1]
    body = functools.partial(
        _fp_body,
        n_layers=len(layers),
        has_p1=has_p1,
        has_cls=cls is not None,
        S=S,
    )
    return pl.pallas_call(
        body,
        grid=(B, N // Nblk),
        in_specs=specs,
        out_specs=pl.BlockSpec((1, Nblk, Cout), lambda b, n: (b, n, 0)),
        out_shape=jax.ShapeDtypeStruct((B, N, Cout), jnp.float32),
    )(*args)


def _pad_lanes(x, mult):
    c = x.shape[-1]
    pad = (-c) % mult
    if pad == 0:
        return x
    return jnp.concatenate(
        [x, jnp.zeros(x.shape[:-1] + (pad,), x.dtype)], axis=-1
    )


def _coords_rows(X, Y, Z):
    B, N = X.shape
    return jnp.concatenate(
        [X[:, None], Y[:, None], Z[:, None], jnp.zeros((B, 5, N), X.dtype)], axis=1
    )


def _coords_cols(X, Y, Z):
    B, N = X.shape
    return jnp.concatenate(
        [X[..., None], Y[..., None], Z[..., None], jnp.zeros((B, N, 5), X.dtype)],
        axis=-1,
    )


def _sa_level(X, Y, Z, xyzc, pts, pc, cfg, branches, Sblk_bq, Sblk_mlp):
    S, (r1, r2), (K1, K2) = cfg
    B, N = X.shape
    Cin = 9 if pts is None else pts.shape[2]

    QX, QY, QZ = _fps(X, Y, Z, S)

    folded = [[_fold(p) for p in br] for br in branches]
    (W1a, b1a), (W2a, b2a), (W3a, b3a) = folded[0]
    (W1b, b1b), (W2b, b2b), (W3b, b3b) = folded[1]
    C1a = W1a.shape[0]
    C1b = W1b.shape[0]

    def wq(W1, C1p):
        return (
            jnp.zeros((8, C1p), jnp.float32)
            .at[:3, : W1.shape[0]]
            .set(W1[:, Cin : Cin + 3].T)
        )

    if pts is None:
        A1, A2 = _atable0(
            pc, W1a.T, b1a.reshape(1, -1), W1b.T, b1b.reshape(1, -1), blk=512
        )
    else:
        A1, A2 = _atable(
            pts.reshape(B * N, Cin),
            xyzc.reshape(B * N, 8),
            W1a[:, :Cin].T,
            wq(W1a, C1a),
            b1a.reshape(1, -1),
            W1b[:, :Cin].T,
            wq(W1b, C1b),
            b1b.reshape(1, -1),
            blk=min(512, B * N),
        )

    P = _coords_rows(X, Y, Z)
    Q = _coords_cols(QX, QY, QZ)
    g1, g2 = _ballq(P, Q, r1, K1, r2, K2, Sblk_bq)

    G1 = _sc_gather(A1, g1.reshape(-1), B * S * K1, C1a).reshape(B, S * K1, C1a)
    G2 = _sc_gather(A2, g2.reshape(-1), B * S * K2, C1b).reshape(B, S * K2, C1b)

    def wpad(W2, C1p):
        return jnp.zeros((C1p, W2.shape[0]), jnp.float32).at[: W2.shape[1]].set(
            W2.T
        )

    o1 = _mlp_max(
        G1, Q, wq(W1a, C1a), wpad(W2a, C1a), b2a.reshape(1, -1),
        W3a.T, b3a.reshape(1, -1), K1, min(Sblk_mlp, S),
    )
    o2 = _mlp_max(
        G2, Q, wq(W1b, C1b), wpad(W2b, C1b), b2b.reshape(1, -1),
        W3b.T, b3b.reshape(1, -1), K2, min(Sblk_mlp, S),
    )
    return QX, QY, QZ, Q, jnp.concatenate([o1, o2], axis=-1)


def kernel(point_cloud, params):
    pc = point_cloud
    B, C0, N0 = pc.shape
    X0 = pc[:, 0, :]
    Y0 = pc[:, 1, :]
    Z0 = pc[:, 2, :]
    x0c = _coords_cols(X0, Y0, Z0)

    QX1, QY1, QZ1, x1c, p1 = _sa_level(
        X0, Y0, Z0, x0c, None, pc, _SA_CFGS[0], params["sa1"],
        Sblk_bq=128, Sblk_mlp=64,
    )
    QX2, QY2, QZ2, x2c, p2 = _sa_level(
        QX1, QY1, QZ1, x1c, p1, None, _SA_CFGS[1], params["sa2"],
        Sblk_bq=128, Sblk_mlp=64,
    )
    QX3, QY3, QZ3, x3c, p3 = _sa_level(
        QX2, QY2, QZ2, x2c, p2, None, _SA_CFGS[2], params["sa3"],
        Sblk_bq=64, Sblk_mlp=64,
    )
    QX4, QY4, QZ4, x4c, p4 = _sa_level(
        QX3, QY3, QZ3, x3c, p3, None, _SA_CFGS[3], params["sa4"],
        Sblk_bq=16, Sblk_mlp=16,
    )

    def fold_layers(ps):
        out = []
        for p in ps:
            W, b = _fold(p)
            out.append((W.T, b))
        return out

    fp1l = fold_layers(params["fp1"])
    fp2l = fold_layers(params["fp2"])
    fp3l = fold_layers(params["fp3"])
    fp4l = fold_layers(params["fp4"])
    Wc1, bc1 = _fold(params["cls1"])
    cls = (
        Wc1.T,
        bc1.reshape(1, -1),
        params["cls2"]["W"].T,
        params["cls2"]["b"].reshape(1, -1),
    )

    u1 = _fp(x3c, x4c, p4, p3, fp1l, None, Nblk=64)
    u2 = _fp(x2c, x3c, u1, p2, fp2l, None, Nblk=128)
    u3 = _fp(x1c, x2c, u2, p1, fp3l, None, Nblk=256)
    pred = _fp(x0c, x1c, u3, None, fp4l, cls, Nblk=256)

    p4_out = jnp.transpose(p4, (0, 2, 1))
    return pred, p4_out


# speedup vs baseline: 28.2110x; 1.0013x over previous
import functools

import jax
import jax.numpy as jnp
import numpy as np
from jax import lax
from jax.experimental import pallas as pl
from jax.experimental.pallas import tpu as pltpu
from jax.experimental.pallas import tpu_sc as plsc

_SA_CFGS = [
    (1024, [0.05, 0.1], [16, 32]),
    (256, [0.1, 0.2], [16, 32]),
    (64, [0.2, 0.4], [16, 32]),
    (16, [0.4, 0.8], [16, 32]),
]
_B = 8
_N0 = 4096


def _fold(p):
    s = p["gamma"] / np.sqrt(1.0 + 1e-5)
    return p["W"] * s[:, None], p["b"] * s + p["beta"]


def _fps_body(x_ref, y_ref, z_ref, qx_ref, qy_ref, qz_ref):
    B, N = x_ref.shape
    S = qx_ref.shape[0]
    X = x_ref[...]
    Y = y_ref[...]
    Z = z_ref[...]
    iota = lax.broadcasted_iota(jnp.int32, (B, N), 1)
    eye = lax.broadcasted_iota(jnp.int32, (B, B), 0) == lax.broadcasted_iota(
        jnp.int32, (B, B), 1
    )

    def row(v):
        return jnp.sum(jnp.where(eye, v, 0.0), axis=0, keepdims=True)

    def body(i, state):
        far, dist = state
        onehot = iota == far
        cx = jnp.sum(jnp.where(onehot, X, 0.0), axis=1, keepdims=True)
        cy = jnp.sum(jnp.where(onehot, Y, 0.0), axis=1, keepdims=True)
        cz = jnp.sum(jnp.where(onehot, Z, 0.0), axis=1, keepdims=True)
        dx = X - cx
        dy = Y - cy
        dz = Z - cz
        d = dx * dx + dy * dy + dz * dz
        dist = jnp.minimum(dist, d)
        m = jnp.max(dist, axis=1, keepdims=True)
        far_new = jnp.min(
            jnp.where(dist == m, iota, N), axis=1, keepdims=True
        ).astype(jnp.int32)
        qx_ref[pl.ds(i, 1), :] = row(cx)
        qy_ref[pl.ds(i, 1), :] = row(cy)
        qz_ref[pl.ds(i, 1), :] = row(cz)
        return far_new, dist

    lax.fori_loop(
        0,
        S,
        body,
        (jnp.zeros((B, 1), jnp.int32), jnp.full((B, N), 1e10, jnp.float32)),
    )


def _fps(X, Y, Z, S):
    B, N = X.shape
    out = jax.ShapeDtypeStruct((S, B), jnp.float32)
    qx, qy, qz = pl.pallas_call(
        _fps_body,
        out_shape=(out, out, out),
    )(X, Y, Z)
    return qx.T, qy.T, qz.T


def _ballq_body(
    p_ref, q_ref, o1_ref, o2_ref, v1_ref, v2_ref, m1_ref, m2_ref, *, N, r1, r2, K1, K2
):
    P = p_ref[0]
    Q = q_ref[0]
    Sblk = Q.shape[0]
    pn = jnp.sum(P * P, axis=0, keepdims=True)
    qn = jnp.sum(Q * Q, axis=1, keepdims=True)
    cross = jnp.dot(Q, P, preferred_element_type=jnp.float32)
    d = qn + pn - 2.0 * cross
    iota32 = lax.broadcasted_iota(jnp.int32, (Sblk, N), 1)
    n32 = jnp.int32(N)
    v1_ref[...] = jnp.where(d <= r1 * r1, iota32, n32)
    v2_ref[...] = jnp.where(d <= r2 * r2, iota32, n32)
    off = pl.program_id(0) * N

    for K, v_ref, m_ref, o_ref in (
        (K1, v1_ref, m1_ref, o1_ref),
        (K2, v2_ref, m2_ref, o2_ref),
    ):
        v0 = v_ref[...]
        maxc = jnp.max(jnp.sum((v0 < n32).astype(jnp.int32), axis=1))
        first = jnp.min(v0, axis=1, keepdims=True)
        firstc = jnp.minimum(first, N - 1) + off
        o_ref[0] = jnp.broadcast_to(firstc, (Sblk, K))
        m_ref[...] = first
        for k in range(1, K):

            @pl.when(k < maxc)
            def _(k=k, o_ref=o_ref, v_ref=v_ref, m_ref=m_ref, firstc=firstc):
                v = v_ref[...]
                m = jnp.min(
                    jnp.where(v > m_ref[...], v, n32), axis=1, keepdims=True
                )
                col = jnp.where(m == n32, firstc, m + off)
                o_ref[0, :, k : k + 1] = col
                m_ref[...] = m


def _ballq(P, Q, r1, K1, r2, K2, Sblk):
    B, _, N = P.shape
    S = Q.shape[1]
    body = functools.partial(_ballq_body, N=N, r1=r1, r2=r2, K1=K1, K2=K2)
    g1, g2 = pl.pallas_call(
        body,
        grid=(B, S // Sblk),
        in_specs=[
            pl.BlockSpec((1, 8, N), lambda b, s: (b, 0, 0)),
            pl.BlockSpec((1, Sblk, 8), lambda b, s: (b, s, 0)),
        ],
        out_specs=(
            pl.BlockSpec((1, Sblk, K1), lambda b, s: (b, s, 0)),
            pl.BlockSpec((1, Sblk, K2), lambda b, s: (b, s, 0)),
        ),
        out_shape=(
            jax.ShapeDtypeStruct((B, S, K1), jnp.int32),
            jax.ShapeDtypeStruct((B, S, K2), jnp.int32),
        ),
        scratch_shapes=[
            pltpu.VMEM((Sblk, N), jnp.int32),
            pltpu.VMEM((Sblk, N), jnp.int32),
            pltpu.VMEM((Sblk, 1), jnp.int32),
            pltpu.VMEM((Sblk, 1), jnp.int32),
        ],
        compiler_params=pltpu.CompilerParams(
            dimension_semantics=("parallel", "parallel")
        ),
    )(P, Q)
    return g1, g2


def _atable0_body(pc_ref, w1_ref, b1_ref, w2_ref, b2_ref, a1_ref, a2_ref):
    P = pc_ref[0]
    Ft = jnp.concatenate([P, P[0:3]], axis=0)
    dn = (((0,), (0,)), ((), ()))
    a1_ref[...] = (
        lax.dot_general(Ft, w1_ref[...], dn, preferred_element_type=jnp.float32)
        + b1_ref[...]
    )
    a2_ref[...] = (
        lax.dot_general(Ft, w2_ref[...], dn, preferred_element_type=jnp.float32)
        + b2_ref[...]
    )


def _atable0(pc, W1, b1, W2, b2, blk):
    B, _, N = pc.shape
    C1 = W1.shape[1]
    C2 = W2.shape[1]
    nb = N // blk
    return pl.pallas_call(
        _atable0_body,
        grid=(B, nb),
        in_specs=[
            pl.BlockSpec((1, 9, blk), lambda b, n: (b, 0, n)),
            pl.BlockSpec((12, C1), lambda b, n: (0, 0)),
            pl.BlockSpec((1, C1), lambda b, n: (0, 0)),
            pl.BlockSpec((12, C2), lambda b, n: (0, 0)),
            pl.BlockSpec((1, C2), lambda b, n: (0, 0)),
        ],
        out_specs=(
            pl.BlockSpec((blk, C1), lambda b, n: (b * nb + n, 0)),
            pl.BlockSpec((blk, C2), lambda b, n: (b * nb + n, 0)),
        ),
        out_shape=(
            jax.ShapeDtypeStruct((B * N, C1), jnp.float32),
            jax.ShapeDtypeStruct((B * N, C2), jnp.float32),
        ),
    )(pc, W1, b1, W2, b2)


def _atable_body(p_ref, x_ref, wp1_ref, wx1_ref, b1_ref, wp2_ref, wx2_ref, b2_ref,
                 a1_ref, a2_ref):
    P = p_ref[...]
    X = x_ref[...]
    a1_ref[...] = (
        jnp.dot(P, wp1_ref[...], preferred_element_type=jnp.float32)
        + jnp.dot(X, wx1_ref[...], preferred_element_type=jnp.float32)
        + b1_ref[...]
    )
    a2_ref[...] = (
        jnp.dot(P, wp2_ref[...], preferred_element_type=jnp.float32)
        + jnp.dot(X, wx2_ref[...], preferred_element_type=jnp.float32)
        + b2_ref[...]
    )


def _atable(pts, xyzc, Wp1, Wx1, b1, Wp2, Wx2, b2, blk):
    R, Cin = pts.shape
    C1 = Wp1.shape[1]
    C2 = Wp2.shape[1]
    return pl.pallas_call(
        _atable_body,
        grid=(R // blk,),
        in_specs=[
            pl.BlockSpec((blk, Cin), lambda i: (i, 0)),
            pl.BlockSpec((blk, 8), lambda i: (i, 0)),
            pl.BlockSpec((Cin, C1), lambda i: (0, 0)),
            pl.BlockSpec((8, C1), lambda i: (0, 0)),
            pl.BlockSpec((1, C1), lambda i: (0, 0)),
            pl.BlockSpec((Cin, C2), lambda i: (0, 0)),
            pl.BlockSpec((8, C2), lambda i: (0, 0)),
            pl.BlockSpec((1, C2), lambda i: (0, 0)),
        ],
        out_specs=(
            pl.BlockSpec((blk, C1), lambda i: (i, 0)),
            pl.BlockSpec((blk, C2), lambda i: (i, 0)),
        ),
        out_shape=(
            jax.ShapeDtypeStruct((R, C1), jnp.float32),
            jax.ShapeDtypeStruct((R, C2), jnp.float32),
        ),
    )(pts, xyzc, Wp1, Wx1, b1, Wp2, Wx2, b2)


def _sc_gather(table, idx, R, C):
    info = plsc.get_sparse_core_info()
    NW = info.num_cores * info.num_subcores
    per_w = R // NW
    chunk = min(128, per_w)
    n_chunks = per_w // chunk
    G = max(1, min(n_chunks, 96000 // (chunk * C), 16))
    groups = []
    j = 0
    while j < n_chunks:
        groups.append((j, min(G, n_chunks - j)))
        j += G
    mesh = plsc.VectorSubcoreMesh(core_axis_name="c", subcore_axis_name="s")

    @functools.partial(
        pl.kernel,
        mesh=mesh,
        out_type=jax.ShapeDtypeStruct((R, C), jnp.float32),
        compiler_params=pltpu.CompilerParams(use_tc_tiling_on_sc=False),
        scratch_types=[
            pltpu.VMEM((n_chunks, chunk), jnp.int32),
            pltpu.VMEM((G * chunk, C), jnp.float32),
            pltpu.SemaphoreType.DMA,
        ],
    )
    def k(table_hbm, idx_hbm, out_hbm, idx_v, rows_v, sem):
        wid = lax.axis_index("s") * info.num_cores + lax.axis_index("c")
        base = wid * per_w
        pltpu.sync_copy(
            idx_hbm.at[pl.ds(wid * n_chunks, n_chunks), :], idx_v
        )
        for g0, glen in groups:
            copies = []
            for t in range(glen):
                copies.append(
                    pltpu.async_copy(
                        table_hbm.at[idx_v.at[g0 + t]],
                        rows_v.at[pl.ds(t * chunk, chunk), :],
                        sem,
                    )
                )
            for c in copies:
                c.wait()
            pltpu.sync_copy(
                rows_v.at[pl.ds(0, glen * chunk), :],
                out_hbm.at[pl.ds(base + g0 * chunk, glen * chunk), :],
            )

    idx2d = idx.reshape(NW * n_chunks, chunk)
    return k(table, idx2d)


def _mlp_max_body(g_ref, q_ref, wq_ref, w2_ref, b2_ref, w3_ref, b3_ref, o_ref, *, K):
    G = g_ref[0]
    q = q_ref[0]
    Sblk = q.shape[0]
    C1 = G.shape[1]
    bq = jnp.dot(q, wq_ref[...], preferred_element_type=jnp.float32)
    H = jnp.maximum(G.reshape(Sblk, K, C1) - bq[:, None, :], 0.0)
    H = H.reshape(Sblk * K, C1)
    H = jnp.maximum(
        jnp.dot(H, w2_ref[...], preferred_element_type=jnp.float32) + b2_ref[...], 0.0
    )
    H = jnp.maximum(
        jnp.dot(H, w3_ref[...], preferred_element_type=jnp.float32) + b3_ref[...], 0.0
    )
    C3 = H.shape[1]
    o_ref[0] = jnp.max(H.reshape(Sblk, K, C3), axis=1)


def _mlp_max(Gf, Q, Wq, W2, b2, W3, b3, K, Sblk):
    B, SK, C1 = Gf.shape
    S = SK // K
    C2 = W2.shape[1]
    C3 = W3.shape[1]
    body = functools.partial(_mlp_max_body, K=K)
    return pl.pallas_call(
        body,
        grid=(B, S // Sblk),
        in_specs=[
            pl.BlockSpec((1, Sblk * K, C1), lambda b, s: (b, s, 0)),
            pl.BlockSpec((1, Sblk, 8), lambda b, s: (b, s, 0)),
            pl.BlockSpec((8, C1), lambda b, s: (0, 0)),
            pl.BlockSpec((C1, C2), lambda b, s: (0, 0)),
            pl.BlockSpec((1, C2), lambda b, s: (0, 0)),
            pl.BlockSpec((C2, C3), lambda b, s: (0, 0)),
            pl.BlockSpec((1, C3), lambda b, s: (0, 0)),
        ],
        out_specs=pl.BlockSpec((1, Sblk, C3), lambda b, s: (b, s, 0)),
        out_shape=jax.ShapeDtypeStruct((B, S, C3), jnp.float32),
        compiler_params=pltpu.CompilerParams(
            dimension_semantics=("parallel", "parallel")
        ),
    )(Gf, Q, Wq, W2, b2, W3, b3)


def _fp_body(x1_ref, x2_ref, p2_ref, *rest, n_layers, has_p1, has_cls, S):
    i = 0
    p1_ref = None
    if has_p1:
        p1_ref = rest[0]
        i = 1
    w_int = rest[i]
    w_p1 = rest[i + 1] if has_p1 else None
    b_1 = rest[i + 1 + (1 if has_p1 else 0)]
    rest = rest[i + 2 + (1 if has_p1 else 0) :]
    layer_ws = []
    for _ in range(n_layers - 1):
        layer_ws.append((rest[0], rest[1]))
        rest = rest[2:]
    cls_ws = None
    if has_cls:
        cls_ws = (rest[0], rest[1], rest[2], rest[3])
        rest = rest[4:]
    o_ref = rest[0]

    q = x1_ref[0]
    p = x2_ref[0]
    Nblk = q.shape[0]
    qn = jnp.sum(q * q, axis=1, keepdims=True)
    psq = p * p
    pn = lax.dot_general(
        jnp.ones((1, 8), jnp.float32),
        psq,
        (((1,), (1,)), ((), ())),
        preferred_element_type=jnp.float32,
    )
    cross = lax.dot_general(
        q, p, (((1,), (1,)), ((), ())), preferred_element_type=jnp.float32
    )
    d = qn + pn - 2.0 * cross
    iota = lax.broadcasted_iota(jnp.int32, (Nblk, S), 1)
    vals = d
    rs = []
    ohs = []
    for _ in range(3):
        m = jnp.min(vals, axis=1, keepdims=True)
        isel = jnp.min(jnp.where(vals == m, iota, S), axis=1, keepdims=True)
        oh = iota == isel
        rs.append(1.0 / (m + 1e-8))
        ohs.append(oh)
        vals = jnp.where(oh, 1e30, vals)
    tot = rs[0] + rs[1] + rs[2]
    Wm = (
        jnp.where(ohs[0], rs[0] / tot, 0.0)
        + jnp.where(ohs[1], rs[1] / tot, 0.0)
        + jnp.where(ohs[2], rs[2] / tot, 0.0)
    )
    interp = jnp.dot(Wm, p2_ref[0], preferred_element_type=jnp.float32)
    acc = jnp.dot(interp, w_int[...], preferred_element_type=jnp.float32)
    if has_p1:
        acc = acc + jnp.dot(
            p1_ref[0], w_p1[...], preferred_element_type=jnp.float32
        )
    H = jnp.maximum(acc + b_1[...], 0.0)
    for w, b in layer_ws:
        H = jnp.maximum(
            jnp.dot(H, w[...], preferred_element_type=jnp.float32) + b[...], 0.0
        )
    if has_cls:
        wc1, bc1, wc2, bc2 = cls_ws
        H = jnp.maximum(
            jnp.dot(H, wc1[...], preferred_element_type=jnp.float32) + bc1[...], 0.0
        )
        logits = jnp.dot(H, wc2[...], preferred_element_type=jnp.float32) + bc2[...]
        mx = jnp.max(logits, axis=1, keepdims=True)
        lse = jnp.log(jnp.sum(jnp.exp(logits - mx), axis=1, keepdims=True)) + mx
        o_ref[0] = logits - lse
    else:
        o_ref[0] = H


def _fp(X1, X2, P2, P1, layers, cls, Nblk):
    B, N, _ = X1.shape
    S = X2.shape[1]
    C2 = P2.shape[2]
    has_p1 = P1 is not None
    C1 = P1.shape[2] if has_p1 else 0

    (W1, b1) = layers[0]
    w_int = W1[C1:, :]
    w_p1 = W1[:C1, :] if has_p1 else None

    args = [X1, X2, P2]
    specs = [
        pl.BlockSpec((1, Nblk, 8), lambda b, n: (b, n, 0)),
        pl.BlockSpec((1, S, 8), lambda b, n: (b, 0, 0)),
        pl.BlockSpec((1, S, C2), lambda b, n: (b, 0, 0)),
    ]
    if has_p1:
        args.append(P1)
        specs.append(pl.BlockSpec((1, Nblk, C1), lambda b, n: (b, n, 0)))
    args.append(w_int)
    specs.append(pl.BlockSpec(w_int.shape, lambda b, n: (0, 0)))
    if has_p1:
        args.append(w_p1)
        specs.append(pl.BlockSpec(w_p1.shape, lambda b, n: (0, 0)))
    args.append(b1.reshape(1, -1))
    specs.append(pl.BlockSpec((1, b1.shape[0]), lambda b, n: (0, 0)))
    for W, b in layers[1:]:
        args.append(W)
        specs.append(pl.BlockSpec(W.shape, lambda b, n: (0, 0)))
        args.append(b.reshape(1, -1))
        specs.append(pl.BlockSpec((1, b.shape[0]), lambda b, n: (0, 0)))
    if cls is not None:
        for arr in cls:
            args.append(arr)
            specs.append(pl.BlockSpec(arr.shape, lambda b, n: (0, 0)))
    Cout = 13 if cls is not None else layers[-1][0].shape[1]
    body = functools.partial(
        _fp_body,
        n_layers=len(layers),
        has_p1=has_p1,
        has_cls=cls is not None,
        S=S,
    )
    return pl.pallas_call(
        body,
        grid=(B, N // Nblk),
        in_specs=specs,
        out_specs=pl.BlockSpec((1, Nblk, Cout), lambda b, n: (b, n, 0)),
        out_shape=jax.ShapeDtypeStruct((B, N, Cout), jnp.float32),
        compiler_params=pltpu.CompilerParams(
            dimension_semantics=("parallel", "parallel")
        ),
    )(*args)


def _pad_lanes(x, mult):
    c = x.shape[-1]
    pad = (-c) % mult
    if pad == 0:
        return x
    return jnp.concatenate(
        [x, jnp.zeros(x.shape[:-1] + (pad,), x.dtype)], axis=-1
    )


def _coords_rows(X, Y, Z):
    B, N = X.shape
    return jnp.concatenate(
        [X[:, None], Y[:, None], Z[:, None], jnp.zeros((B, 5, N), X.dtype)], axis=1
    )


def _coords_cols(X, Y, Z):
    B, N = X.shape
    return jnp.concatenate(
        [X[..., None], Y[..., None], Z[..., None], jnp.zeros((B, N, 5), X.dtype)],
        axis=-1,
    )


def _sa_level(X, Y, Z, xyzc, pts, pc, cfg, branches, Sblk_bq, Sblk_mlp):
    S, (r1, r2), (K1, K2) = cfg
    B, N = X.shape
    Cin = 9 if pts is None else pts.shape[2]

    QX, QY, QZ = _fps(X, Y, Z, S)

    folded = [[_fold(p) for p in br] for br in branches]
    (W1a, b1a), (W2a, b2a), (W3a, b3a) = folded[0]
    (W1b, b1b), (W2b, b2b), (W3b, b3b) = folded[1]
    C1a = W1a.shape[0]
    C1b = W1b.shape[0]

    def wq(W1, C1p):
        return (
            jnp.zeros((8, C1p), jnp.float32)
            .at[:3, : W1.shape[0]]
            .set(W1[:, Cin : Cin + 3].T)
        )

    if pts is None:
        A1, A2 = _atable0(
            pc, W1a.T, b1a.reshape(1, -1), W1b.T, b1b.reshape(1, -1), blk=512
        )
    else:
        A1, A2 = _atable(
            pts.reshape(B * N, Cin),
            xyzc.reshape(B * N, 8),
            W1a[:, :Cin].T,
            wq(W1a, C1a),
            b1a.reshape(1, -1),
            W1b[:, :Cin].T,
            wq(W1b, C1b),
            b1b.reshape(1, -1),
            blk=min(512, B * N),
        )

    P = _coords_rows(X, Y, Z)
    Q = _coords_cols(QX, QY, QZ)
    g1, g2 = _ballq(P, Q, r1, K1, r2, K2, Sblk_bq)

    G1 = _sc_gather(A1, g1.reshape(-1), B * S * K1, C1a).reshape(B, S * K1, C1a)
    G2 = _sc_gather(A2, g2.reshape(-1), B * S * K2, C1b).reshape(B, S * K2, C1b)

    def wpad(W2, C1p):
        return jnp.zeros((C1p, W2.shape[0]), jnp.float32).at[: W2.shape[1]].set(
            W2.T
        )

    o1 = _mlp_max(
        G1, Q, wq(W1a, C1a), wpad(W2a, C1a), b2a.reshape(1, -1),
        W3a.T, b3a.reshape(1, -1), K1, min(Sblk_mlp, S),
    )
    o2 = _mlp_max(
        G2, Q, wq(W1b, C1b), wpad(W2b, C1b), b2b.reshape(1, -1),
        W3b.T, b3b.reshape(1, -1), K2, min(Sblk_mlp, S),
    )
    return QX, QY, QZ, Q, jnp.concatenate([o1, o2], axis=-1)


def kernel(point_cloud, params):
    pc = point_cloud
    B, C0, N0 = pc.shape
    X0 = pc[:, 0, :]
    Y0 = pc[:, 1, :]
    Z0 = pc[:, 2, :]
    x0c = _coords_cols(X0, Y0, Z0)

    QX1, QY1, QZ1, x1c, p1 = _sa_level(
        X0, Y0, Z0, x0c, None, pc, _SA_CFGS[0], params["sa1"],
        Sblk_bq=128, Sblk_mlp=64,
    )
    QX2, QY2, QZ2, x2c, p2 = _sa_level(
        QX1, QY1, QZ1, x1c, p1, None, _SA_CFGS[1], params["sa2"],
        Sblk_bq=128, Sblk_mlp=64,
    )
    QX3, QY3, QZ3, x3c, p3 = _sa_level(
        QX2, QY2, QZ2, x2c, p2, None, _SA_CFGS[2], params["sa3"],
        Sblk_bq=64, Sblk_mlp=64,
    )
    QX4, QY4, QZ4, x4c, p4 = _sa_level(
        QX3, QY3, QZ3, x3c, p3, None, _SA_CFGS[3], params["sa4"],
        Sblk_bq=16, Sblk_mlp=16,
    )

    def fold_layers(ps):
        out = []
        for p in ps:
            W, b = _fold(p)
            out.append((W.T, b))
        return out

    fp1l = fold_layers(params["fp1"])
    fp2l = fold_layers(params["fp2"])
    fp3l = fold_layers(params["fp3"])
    fp4l = fold_layers(params["fp4"])
    Wc1, bc1 = _fold(params["cls1"])
    cls = (
        Wc1.T,
        bc1.reshape(1, -1),
        params["cls2"]["W"].T,
        params["cls2"]["b"].reshape(1, -1),
    )

    u1 = _fp(x3c, x4c, p4, p3, fp1l, None, Nblk=64)
    u2 = _fp(x2c, x3c, u1, p2, fp2l, None, Nblk=128)
    u3 = _fp(x1c, x2c, u2, p1, fp3l, None, Nblk=256)
    pred = _fp(x0c, x1c, u3, None, fp4l, cls, Nblk=256)

    p4_out = jnp.transpose(p4, (0, 2, 1))
    return pred, p4_out


# speedup vs baseline: 28.2499x; 1.0014x over previous
import functools

import jax
import jax.numpy as jnp
import numpy as np
from jax import lax
from jax.experimental import pallas as pl
from jax.experimental.pallas import tpu as pltpu
from jax.experimental.pallas import tpu_sc as plsc

_SA_CFGS = [
    (1024, [0.05, 0.1], [16, 32]),
    (256, [0.1, 0.2], [16, 32]),
    (64, [0.2, 0.4], [16, 32]),
    (16, [0.4, 0.8], [16, 32]),
]
def _fold(p):
    s = p["gamma"] / np.sqrt(1.0 + 1e-5)
    return p["W"] * s[:, None], p["b"] * s + p["beta"]


def _fps_body(x_ref, y_ref, z_ref, qx_ref, qy_ref, qz_ref):
    B, N = x_ref.shape
    S = qx_ref.shape[0]
    X = x_ref[...]
    Y = y_ref[...]
    Z = z_ref[...]
    iota = lax.broadcasted_iota(jnp.int32, (B, N), 1)
    eye = lax.broadcasted_iota(jnp.int32, (B, B), 0) == lax.broadcasted_iota(
        jnp.int32, (B, B), 1
    )

    def row(v):
        return jnp.sum(jnp.where(eye, v, 0.0), axis=0, keepdims=True)

    def body(i, state):
        far, dist = state
        onehot = iota == far
        cx = jnp.sum(jnp.where(onehot, X, 0.0), axis=1, keepdims=True)
        cy = jnp.sum(jnp.where(onehot, Y, 0.0), axis=1, keepdims=True)
        cz = jnp.sum(jnp.where(onehot, Z, 0.0), axis=1, keepdims=True)
        dx = X - cx
        dy = Y - cy
        dz = Z - cz
        d = dx * dx + dy * dy + dz * dz
        dist = jnp.minimum(dist, d)
        m = jnp.max(dist, axis=1, keepdims=True)
        far_new = jnp.min(
            jnp.where(dist == m, iota, N), axis=1, keepdims=True
        ).astype(jnp.int32)
        qx_ref[pl.ds(i, 1), :] = row(cx)
        qy_ref[pl.ds(i, 1), :] = row(cy)
        qz_ref[pl.ds(i, 1), :] = row(cz)
        return far_new, dist

    lax.fori_loop(
        0,
        S,
        body,
        (jnp.zeros((B, 1), jnp.int32), jnp.full((B, N), 1e10, jnp.float32)),
    )


def _fps(X, Y, Z, S):
    B, N = X.shape
    out = jax.ShapeDtypeStruct((S, B), jnp.float32)
    qx, qy, qz = pl.pallas_call(
        _fps_body,
        out_shape=(out, out, out),
    )(X, Y, Z)
    return qx.T, qy.T, qz.T


def _ballq_body(
    p_ref, q_ref, o1_ref, o2_ref, v1_ref, v2_ref, m1_ref, m2_ref, *, N, r1, r2, K1, K2
):
    P = p_ref[0]
    Q = q_ref[0]
    Sblk = Q.shape[0]
    pn = jnp.sum(P * P, axis=0, keepdims=True)
    qn = jnp.sum(Q * Q, axis=1, keepdims=True)
    cross = jnp.dot(Q, P, preferred_element_type=jnp.float32)
    d = qn + pn - 2.0 * cross
    iota32 = lax.broadcasted_iota(jnp.int32, (Sblk, N), 1)
    n32 = jnp.int32(N)
    v1_ref[...] = jnp.where(d <= r1 * r1, iota32, n32)
    v2_ref[...] = jnp.where(d <= r2 * r2, iota32, n32)
    off = pl.program_id(0) * N

    for K, v_ref, m_ref, o_ref in (
        (K1, v1_ref, m1_ref, o1_ref),
        (K2, v2_ref, m2_ref, o2_ref),
    ):
        v0 = v_ref[...]
        maxc = jnp.max(jnp.sum((v0 < n32).astype(jnp.int32), axis=1))
        first = jnp.min(v0, axis=1, keepdims=True)
        firstc = jnp.minimum(first, N - 1) + off
        o_ref[0] = jnp.broadcast_to(firstc, (Sblk, K))
        m_ref[...] = first
        for k in range(1, K):

            @pl.when(k < maxc)
            def _(k=k, o_ref=o_ref, v_ref=v_ref, m_ref=m_ref, firstc=firstc):
                v = v_ref[...]
                m = jnp.min(
                    jnp.where(v > m_ref[...], v, n32), axis=1, keepdims=True
                )
                col = jnp.where(m == n32, firstc, m + off)
                o_ref[0, :, k : k + 1] = col
                m_ref[...] = m


def _ballq(P, Q, r1, K1, r2, K2, Sblk):
    B, _, N = P.shape
    S = Q.shape[1]
    body = functools.partial(_ballq_body, N=N, r1=r1, r2=r2, K1=K1, K2=K2)
    g1, g2 = pl.pallas_call(
        body,
        grid=(B, S // Sblk),
        in_specs=[
            pl.BlockSpec((1, 8, N), lambda b, s: (b, 0, 0)),
            pl.BlockSpec((1, Sblk, 8), lambda b, s: (b, s, 0)),
        ],
        out_specs=(
            pl.BlockSpec((1, Sblk, K1), lambda b, s: (b, s, 0)),
            pl.BlockSpec((1, Sblk, K2), lambda b, s: (b, s, 0)),
        ),
        out_shape=(
            jax.ShapeDtypeStruct((B, S, K1), jnp.int32),
            jax.ShapeDtypeStruct((B, S, K2), jnp.int32),
        ),
        scratch_shapes=[
            pltpu.VMEM((Sblk, N), jnp.int32),
            pltpu.VMEM((Sblk, N), jnp.int32),
            pltpu.VMEM((Sblk, 1), jnp.int32),
            pltpu.VMEM((Sblk, 1), jnp.int32),
        ],
        compiler_params=pltpu.CompilerParams(
            dimension_semantics=("parallel", "parallel")
        ),
    )(P, Q)
    return g1, g2


def _atable0_body(pc_ref, w1_ref, b1_ref, w2_ref, b2_ref, a1_ref, a2_ref):
    P = pc_ref[0]
    Ft = jnp.concatenate([P, P[0:3]], axis=0)
    dn = (((0,), (0,)), ((), ()))
    a1_ref[...] = (
        lax.dot_general(Ft, w1_ref[...], dn, preferred_element_type=jnp.float32)
        + b1_ref[...]
    )
    a2_ref[...] = (
        lax.dot_general(Ft, w2_ref[...], dn, preferred_element_type=jnp.float32)
        + b2_ref[...]
    )


def _atable0(pc, W1, b1, W2, b2, blk):
    B, _, N = pc.shape
    C1 = W1.shape[1]
    C2 = W2.shape[1]
    nb = N // blk
    return pl.pallas_call(
        _atable0_body,
        grid=(B, nb),
        in_specs=[
            pl.BlockSpec((1, 9, blk), lambda b, n: (b, 0, n)),
            pl.BlockSpec((12, C1), lambda b, n: (0, 0)),
            pl.BlockSpec((1, C1), lambda b, n: (0, 0)),
            pl.BlockSpec((12, C2), lambda b, n: (0, 0)),
            pl.BlockSpec((1, C2), lambda b, n: (0, 0)),
        ],
        out_specs=(
            pl.BlockSpec((blk, C1), lambda b, n: (b * nb + n, 0)),
            pl.BlockSpec((blk, C2), lambda b, n: (b * nb + n, 0)),
        ),
        out_shape=(
            jax.ShapeDtypeStruct((B * N, C1), jnp.float32),
            jax.ShapeDtypeStruct((B * N, C2), jnp.float32),
        ),
    )(pc, W1, b1, W2, b2)


def _atable_body(p_ref, x_ref, wp1_ref, wx1_ref, b1_ref, wp2_ref, wx2_ref, b2_ref,
                 a1_ref, a2_ref):
    P = p_ref[...]
    X = x_ref[...]
    a1_ref[...] = (
        jnp.dot(P, wp1_ref[...], preferred_element_type=jnp.float32)
        + jnp.dot(X, wx1_ref[...], preferred_element_type=jnp.float32)
        + b1_ref[...]
    )
    a2_ref[...] = (
        jnp.dot(P, wp2_ref[...], preferred_element_type=jnp.float32)
        + jnp.dot(X, wx2_ref[...], preferred_element_type=jnp.float32)
        + b2_ref[...]
    )


def _atable(pts, xyzc, Wp1, Wx1, b1, Wp2, Wx2, b2, blk):
    R, Cin = pts.shape
    C1 = Wp1.shape[1]
    C2 = Wp2.shape[1]
    return pl.pallas_call(
        _atable_body,
        grid=(R // blk,),
        in_specs=[
            pl.BlockSpec((blk, Cin), lambda i: (i, 0)),
            pl.BlockSpec((blk, 8), lambda i: (i, 0)),
            pl.BlockSpec((Cin, C1), lambda i: (0, 0)),
            pl.BlockSpec((8, C1), lambda i: (0, 0)),
            pl.BlockSpec((1, C1), lambda i: (0, 0)),
            pl.BlockSpec((Cin, C2), lambda i: (0, 0)),
            pl.BlockSpec((8, C2), lambda i: (0, 0)),
            pl.BlockSpec((1, C2), lambda i: (0, 0)),
        ],
        out_specs=(
            pl.BlockSpec((blk, C1), lambda i: (i, 0)),
            pl.BlockSpec((blk, C2), lambda i: (i, 0)),
        ),
        out_shape=(
            jax.ShapeDtypeStruct((R, C1), jnp.float32),
            jax.ShapeDtypeStruct((R, C2), jnp.float32),
        ),
    )(pts, xyzc, Wp1, Wx1, b1, Wp2, Wx2, b2)


def _sc_gather(table, idx, R, C):
    info = plsc.get_sparse_core_info()
    NW = info.num_cores * info.num_subcores
    per_w = R // NW
    chunk = min(128, per_w)
    n_chunks = per_w // chunk
    G = max(1, min(n_chunks, 96000 // (chunk * C), 16))
    groups = []
    j = 0
    while j < n_chunks:
        groups.append((j, min(G, n_chunks - j)))
        j += G
    mesh = plsc.VectorSubcoreMesh(core_axis_name="c", subcore_axis_name="s")

    @functools.partial(
        pl.kernel,
        mesh=mesh,
        out_type=jax.ShapeDtypeStruct((R, C), jnp.float32),
        compiler_params=pltpu.CompilerParams(use_tc_tiling_on_sc=False),
        scratch_types=[
            pltpu.VMEM((n_chunks, chunk), jnp.int32),
            pltpu.VMEM((G * chunk, C), jnp.float32),
            pltpu.SemaphoreType.DMA,
        ],
    )
    def k(table_hbm, idx_hbm, out_hbm, idx_v, rows_v, sem):
        wid = lax.axis_index("s") * info.num_cores + lax.axis_index("c")
        base = wid * per_w
        pltpu.sync_copy(
            idx_hbm.at[pl.ds(wid * n_chunks, n_chunks), :], idx_v
        )
        for g0, glen in groups:
            copies = []
            for t in range(glen):
                copies.append(
                    pltpu.async_copy(
                        table_hbm.at[idx_v.at[g0 + t]],
                        rows_v.at[pl.ds(t * chunk, chunk), :],
                        sem,
                    )
                )
            for c in copies:
                c.wait()
            pltpu.sync_copy(
                rows_v.at[pl.ds(0, glen * chunk), :],
                out_hbm.at[pl.ds(base + g0 * chunk, glen * chunk), :],
            )

    idx2d = idx.reshape(NW * n_chunks, chunk)
    return k(table, idx2d)


def _mlp_max_body(g_ref, q_ref, wq_ref, w2_ref, b2_ref, w3_ref, b3_ref, o_ref, *, K):
    G = g_ref[0]
    q = q_ref[0]
    Sblk = q.shape[0]
    C1 = G.shape[1]
    bq = jnp.dot(q, wq_ref[...], preferred_element_type=jnp.float32)
    H = jnp.maximum(G.reshape(Sblk, K, C1) - bq[:, None, :], 0.0)
    H = H.reshape(Sblk * K, C1)
    H = jnp.maximum(
        jnp.dot(H, w2_ref[...], preferred_element_type=jnp.float32) + b2_ref[...], 0.0
    )
    H = jnp.maximum(
        jnp.dot(H, w3_ref[...], preferred_element_type=jnp.float32) + b3_ref[...], 0.0
    )
    C3 = H.shape[1]
    o_ref[0] = jnp.max(H.reshape(Sblk, K, C3), axis=1)


def _mlp_max(Gf, Q, Wq, W2, b2, W3, b3, K, Sblk):
    B, SK, C1 = Gf.shape
    S = SK // K
    C2 = W2.shape[1]
    C3 = W3.shape[1]
    body = functools.partial(_mlp_max_body, K=K)
    return pl.pallas_call(
        body,
        grid=(B, S // Sblk),
        in_specs=[
            pl.BlockSpec((1, Sblk * K, C1), lambda b, s: (b, s, 0)),
            pl.BlockSpec((1, Sblk, 8), lambda b, s: (b, s, 0)),
            pl.BlockSpec((8, C1), lambda b, s: (0, 0)),
            pl.BlockSpec((C1, C2), lambda b, s: (0, 0)),
            pl.BlockSpec((1, C2), lambda b, s: (0, 0)),
            pl.BlockSpec((C2, C3), lambda b, s: (0, 0)),
            pl.BlockSpec((1, C3), lambda b, s: (0, 0)),
        ],
        out_specs=pl.BlockSpec((1, Sblk, C3), lambda b, s: (b, s, 0)),
        out_shape=jax.ShapeDtypeStruct((B, S, C3), jnp.float32),
        compiler_params=pltpu.CompilerParams(
            dimension_semantics=("parallel", "parallel")
        ),
    )(Gf, Q, Wq, W2, b2, W3, b3)


def _fp_body(x1_ref, x2_ref, p2_ref, *rest, n_layers, has_p1, has_cls, S):
    i = 0
    p1_ref = None
    if has_p1:
        p1_ref = rest[0]
        i = 1
    w_int = rest[i]
    w_p1 = rest[i + 1] if has_p1 else None
    b_1 = rest[i + 1 + (1 if has_p1 else 0)]
    rest = rest[i + 2 + (1 if has_p1 else 0) :]
    layer_ws = []
    for _ in range(n_layers - 1):
        layer_ws.append((rest[0], rest[1]))
        rest = rest[2:]
    cls_ws = None
    if has_cls:
        cls_ws = (rest[0], rest[1], rest[2], rest[3])
        rest = rest[4:]
    o_ref = rest[0]

    q = x1_ref[0]
    p = x2_ref[0]
    Nblk = q.shape[0]
    qn = jnp.sum(q * q, axis=1, keepdims=True)
    psq = p * p
    pn = lax.dot_general(
        jnp.ones((1, 8), jnp.float32),
        psq,
        (((1,), (1,)), ((), ())),
        preferred_element_type=jnp.float32,
    )
    cross = lax.dot_general(
        q, p, (((1,), (1,)), ((), ())), preferred_element_type=jnp.float32
    )
    d = qn + pn - 2.0 * cross
    iota = lax.broadcasted_iota(jnp.int32, (Nblk, S), 1)
    vals = d
    rs = []
    ohs = []
    for _ in range(3):
        m = jnp.min(vals, axis=1, keepdims=True)
        isel = jnp.min(jnp.where(vals == m, iota, S), axis=1, keepdims=True)
        oh = iota == isel
        rs.append(1.0 / (m + 1e-8))
        ohs.append(oh)
        vals = jnp.where(oh, 1e30, vals)
    tot = rs[0] + rs[1] + rs[2]
    Wm = (
        jnp.where(ohs[0], rs[0] / tot, 0.0)
        + jnp.where(ohs[1], rs[1] / tot, 0.0)
        + jnp.where(ohs[2], rs[2] / tot, 0.0)
    )
    interp = jnp.dot(Wm, p2_ref[0], preferred_element_type=jnp.float32)
    acc = jnp.dot(interp, w_int[...], preferred_element_type=jnp.float32)
    if has_p1:
        acc = acc + jnp.dot(
            p1_ref[0], w_p1[...], preferred_element_type=jnp.float32
        )
    H = jnp.maximum(acc + b_1[...], 0.0)
    for w, b in layer_ws:
        H = jnp.maximum(
            jnp.dot(H, w[...], preferred_element_type=jnp.float32) + b[...], 0.0
        )
    if has_cls:
        wc1, bc1, wc2, bc2 = cls_ws
        H = jnp.maximum(
            jnp.dot(H, wc1[...], preferred_element_type=jnp.float32) + bc1[...], 0.0
        )
        logits = jnp.dot(H, wc2[...], preferred_element_type=jnp.float32) + bc2[...]
        mx = jnp.max(logits, axis=1, keepdims=True)
        lse = jnp.log(jnp.sum(jnp.exp(logits - mx), axis=1, keepdims=True)) + mx
        o_ref[0] = logits - lse
    else:
        o_ref[0] = H


def _fp(X1, X2, P2, P1, layers, cls, Nblk):
    B, N, _ = X1.shape
    S = X2.shape[1]
    C2 = P2.shape[2]
    has_p1 = P1 is not None
    C1 = P1.shape[2] if has_p1 else 0

    (W1, b1) = layers[0]
    w_int = W1[C1:, :]
    w_p1 = W1[:C1, :] if has_p1 else None

    args = [X1, X2, P2]
    specs = [
        pl.BlockSpec((1, Nblk, 8), lambda b, n: (b, n, 0)),
        pl.BlockSpec((1, S, 8), lambda b, n: (b, 0, 0)),
        pl.BlockSpec((1, S, C2), lambda b, n: (b, 0, 0)),
    ]
    if has_p1:
        args.append(P1)
        specs.append(pl.BlockSpec((1, Nblk, C1), lambda b, n: (b, n, 0)))
    args.append(w_int)
    specs.append(pl.BlockSpec(w_int.shape, lambda b, n: (0, 0)))
    if has_p1:
        args.append(w_p1)
        specs.append(pl.BlockSpec(w_p1.shape, lambda b, n: (0, 0)))
    args.append(b1.reshape(1, -1))
    specs.append(pl.BlockSpec((1, b1.shape[0]), lambda b, n: (0, 0)))
    for W, b in layers[1:]:
        args.append(W)
        specs.append(pl.BlockSpec(W.shape, lambda b, n: (0, 0)))
        args.append(b.reshape(1, -1))
        specs.append(pl.BlockSpec((1, b.shape[0]), lambda b, n: (0, 0)))
    if cls is not None:
        for arr in cls:
            args.append(arr)
            specs.append(pl.BlockSpec(arr.shape, lambda b, n: (0, 0)))
    Cout = 13 if cls is not None else layers[-1][0].shape[1]
    body = functools.partial(
        _fp_body,
        n_layers=len(layers),
        has_p1=has_p1,
        has_cls=cls is not None,
        S=S,
    )
    return pl.pallas_call(
        body,
        grid=(B, N // Nblk),
        in_specs=specs,
        out_specs=pl.BlockSpec((1, Nblk, Cout), lambda b, n: (b, n, 0)),
        out_shape=jax.ShapeDtypeStruct((B, N, Cout), jnp.float32),
        compiler_params=pltpu.CompilerParams(
            dimension_semantics=("parallel", "parallel")
        ),
    )(*args)


def _coords_rows(X, Y, Z):
    B, N = X.shape
    return jnp.concatenate(
        [X[:, None], Y[:, None], Z[:, None], jnp.zeros((B, 5, N), X.dtype)], axis=1
    )


def _coords_cols(X, Y, Z):
    B, N = X.shape
    return jnp.concatenate(
        [X[..., None], Y[..., None], Z[..., None], jnp.zeros((B, N, 5), X.dtype)],
        axis=-1,
    )


def _sa_level(X, Y, Z, xyzc, pts, pc, cfg, branches, Sblk_bq, Sblk_mlp):
    S, (r1, r2), (K1, K2) = cfg
    B, N = X.shape
    Cin = 9 if pts is None else pts.shape[2]

    QX, QY, QZ = _fps(X, Y, Z, S)

    folded = [[_fold(p) for p in br] for br in branches]
    (W1a, b1a), (W2a, b2a), (W3a, b3a) = folded[0]
    (W1b, b1b), (W2b, b2b), (W3b, b3b) = folded[1]
    C1a = W1a.shape[0]
    C1b = W1b.shape[0]

    def wq(W1, C1p):
        return (
            jnp.zeros((8, C1p), jnp.float32)
            .at[:3, : W1.shape[0]]
            .set(W1[:, Cin : Cin + 3].T)
        )

    if pts is None:
        A1, A2 = _atable0(
            pc, W1a.T, b1a.reshape(1, -1), W1b.T, b1b.reshape(1, -1), blk=512
        )
    else:
        A1, A2 = _atable(
            pts.reshape(B * N, Cin),
            xyzc.reshape(B * N, 8),
            W1a[:, :Cin].T,
            wq(W1a, C1a),
            b1a.reshape(1, -1),
            W1b[:, :Cin].T,
            wq(W1b, C1b),
            b1b.reshape(1, -1),
            blk=min(512, B * N),
        )

    P = _coords_rows(X, Y, Z)
    Q = _coords_cols(QX, QY, QZ)
    g1, g2 = _ballq(P, Q, r1, K1, r2, K2, Sblk_bq)

    G1 = _sc_gather(A1, g1.reshape(-1), B * S * K1, C1a).reshape(B, S * K1, C1a)
    G2 = _sc_gather(A2, g2.reshape(-1), B * S * K2, C1b).reshape(B, S * K2, C1b)

    def wpad(W2, C1p):
        return jnp.zeros((C1p, W2.shape[0]), jnp.float32).at[: W2.shape[1]].set(
            W2.T
        )

    o1 = _mlp_max(
        G1, Q, wq(W1a, C1a), wpad(W2a, C1a), b2a.reshape(1, -1),
        W3a.T, b3a.reshape(1, -1), K1, min(Sblk_mlp, S),
    )
    o2 = _mlp_max(
        G2, Q, wq(W1b, C1b), wpad(W2b, C1b), b2b.reshape(1, -1),
        W3b.T, b3b.reshape(1, -1), K2, min(Sblk_mlp, S),
    )
    return QX, QY, QZ, Q, jnp.concatenate([o1, o2], axis=-1)


def kernel(point_cloud, params):
    pc = point_cloud
    B, C0, N0 = pc.shape
    X0 = pc[:, 0, :]
    Y0 = pc[:, 1, :]
    Z0 = pc[:, 2, :]
    x0c = _coords_cols(X0, Y0, Z0)

    QX1, QY1, QZ1, x1c, p1 = _sa_level(
        X0, Y0, Z0, x0c, None, pc, _SA_CFGS[0], params["sa1"],
        Sblk_bq=128, Sblk_mlp=64,
    )
    QX2, QY2, QZ2, x2c, p2 = _sa_level(
        QX1, QY1, QZ1, x1c, p1, None, _SA_CFGS[1], params["sa2"],
        Sblk_bq=128, Sblk_mlp=64,
    )
    QX3, QY3, QZ3, x3c, p3 = _sa_level(
        QX2, QY2, QZ2, x2c, p2, None, _SA_CFGS[2], params["sa3"],
        Sblk_bq=64, Sblk_mlp=64,
    )
    QX4, QY4, QZ4, x4c, p4 = _sa_level(
        QX3, QY3, QZ3, x3c, p3, None, _SA_CFGS[3], params["sa4"],
        Sblk_bq=16, Sblk_mlp=16,
    )

    def fold_layers(ps):
        out = []
        for p in ps:
            W, b = _fold(p)
            out.append((W.T, b))
        return out

    fp1l = fold_layers(params["fp1"])
    fp2l = fold_layers(params["fp2"])
    fp3l = fold_layers(params["fp3"])
    fp4l = fold_layers(params["fp4"])
    Wc1, bc1 = _fold(params["cls1"])
    cls = (
        Wc1.T,
        bc1.reshape(1, -1),
        params["cls2"]["W"].T,
        params["cls2"]["b"].reshape(1, -1),
    )

    u1 = _fp(x3c, x4c, p4, p3, fp1l, None, Nblk=64)
    u2 = _fp(x2c, x3c, u1, p2, fp2l, None, Nblk=128)
    u3 = _fp(x1c, x2c, u2, p1, fp3l, None, Nblk=256)
    pred = _fp(x0c, x1c, u3, None, fp4l, cls, Nblk=256)

    p4_out = jnp.transpose(p4, (0, 2, 1))
    return pred, p4_out
